# baseline (speedup 1.0000x reference)
import functools

import jax
import jax.numpy as jnp
from jax.experimental import pallas as pl
from jax.experimental.pallas import tpu as pltpu

N = 10000
E = 320000
D = 128
ED = 16
H = 128

EB = 1280
NB = 2000


def _silu(v):
    return v * (1.0 / (1.0 + jnp.exp(-v)))


def _proj_body(x_ref, w1a_ref, w1b_ref, b1_ref, xa_ref, xb_ref):
    x = x_ref[...]
    xa_ref[...] = jnp.dot(x, w1a_ref[...], preferred_element_type=jnp.float32) + b1_ref[...]
    xb_ref[...] = jnp.dot(x, w1b_ref[...], preferred_element_type=jnp.float32)


def _proj(x, W1a, W1b, b1):
    grid = (N // NB,)
    return pl.pallas_call(
        _proj_body,
        grid=grid,
        in_specs=[
            pl.BlockSpec((NB, D), lambda i: (i, 0)),
            pl.BlockSpec((D, H), lambda i: (0, 0)),
            pl.BlockSpec((D, H), lambda i: (0, 0)),
            pl.BlockSpec((1, H), lambda i: (0, 0)),
        ],
        out_specs=[
            pl.BlockSpec((NB, H), lambda i: (i, 0)),
            pl.BlockSpec((NB, H), lambda i: (i, 0)),
        ],
        out_shape=[
            jax.ShapeDtypeStruct((N, H), jnp.float32),
            jax.ShapeDtypeStruct((N, H), jnp.float32),
        ],
    )(x, W1a, W1b, b1)


def _edge_body(g_ref, r2_ref, ea_ref, w1e_ref, w1r_ref, w2_ref, b2_ref,
               w5_ref, b5_ref, m_ref, gam_ref):
    pre1 = (g_ref[...]
            + r2_ref[...] * w1r_ref[...]
            + jnp.dot(ea_ref[...], w1e_ref[...], preferred_element_type=jnp.float32))
    z1 = _silu(pre1)
    m = _silu(jnp.dot(z1, w2_ref[...], preferred_element_type=jnp.float32) + b2_ref[...])
    m_ref[...] = m
    gam_ref[...] = jnp.dot(m, w5_ref[...], preferred_element_type=jnp.float32) + b5_ref[...]


def _edge_mlp(g, r2, ea, W1e, w1r, W2, b2, W5, b5):
    grid = (E // EB,)
    return pl.pallas_call(
        _edge_body,
        grid=grid,
        in_specs=[
            pl.BlockSpec((EB, H), lambda i: (i, 0)),
            pl.BlockSpec((EB, 1), lambda i: (i, 0)),
            pl.BlockSpec((EB, ED), lambda i: (i, 0)),
            pl.BlockSpec((ED, H), lambda i: (0, 0)),
            pl.BlockSpec((1, H), lambda i: (0, 0)),
            pl.BlockSpec((H, H), lambda i: (0, 0)),
            pl.BlockSpec((1, H), lambda i: (0, 0)),
            pl.BlockSpec((H, 1), lambda i: (0, 0)),
            pl.BlockSpec((1, 1), lambda i: (0, 0)),
        ],
        out_specs=[
            pl.BlockSpec((EB, H), lambda i: (i, 0)),
            pl.BlockSpec((EB, 1), lambda i: (i, 0)),
        ],
        out_shape=[
            jax.ShapeDtypeStruct((E, H), jnp.float32),
            jax.ShapeDtypeStruct((E, 1), jnp.float32),
        ],
        compiler_params=pltpu.CompilerParams(
            dimension_semantics=("arbitrary",),
        ),
    )(g, r2, ea, W1e, w1r, W2, b2, W5, b5)


def _node_body(x_ref, ms_ref, deg_ref, coord_ref, pos_ref, w3a_ref, w3b_ref,
               b3_ref, w4_ref, b4_ref, xn_ref, pn_ref):
    inv = 1.0 / jnp.maximum(deg_ref[...], 1.0)
    ms = ms_ref[...] * inv
    pre = (jnp.dot(x_ref[...], w3a_ref[...], preferred_element_type=jnp.float32)
           + jnp.dot(ms, w3b_ref[...], preferred_element_type=jnp.float32)
           + b3_ref[...])
    xn_ref[...] = jnp.dot(_silu(pre), w4_ref[...], preferred_element_type=jnp.float32) + b4_ref[...]
    pn_ref[...] = pos_ref[...] + coord_ref[...] * inv


def _node_mlp(x, m_sum, deg, coord, pos, W3a, W3b, b3, W4, b4):
    grid = (N // NB,)
    return pl.pallas_call(
        _node_body,
        grid=grid,
        in_specs=[
            pl.BlockSpec((NB, D), lambda i: (i, 0)),
            pl.BlockSpec((NB, H), lambda i: (i, 0)),
            pl.BlockSpec((NB, 1), lambda i: (i, 0)),
            pl.BlockSpec((NB, 3), lambda i: (i, 0)),
            pl.BlockSpec((NB, 3), lambda i: (i, 0)),
            pl.BlockSpec((D, H), lambda i: (0, 0)),
            pl.BlockSpec((H, H), lambda i: (0, 0)),
            pl.BlockSpec((1, H), lambda i: (0, 0)),
            pl.BlockSpec((H, D), lambda i: (0, 0)),
            pl.BlockSpec((1, D), lambda i: (0, 0)),
        ],
        out_specs=[
            pl.BlockSpec((NB, D), lambda i: (i, 0)),
            pl.BlockSpec((NB, 3), lambda i: (i, 0)),
        ],
        out_shape=[
            jax.ShapeDtypeStruct((N, D), jnp.float32),
            jax.ShapeDtypeStruct((N, 3), jnp.float32),
        ],
    )(x, m_sum, deg, coord, pos, W3a, W3b, b3, W4, b4)


def kernel(x, pos, edge_index, edge_attr, W1, b1, W2, b2, W3, b3, W4, b4, W5, b5):
    src = edge_index[0]
    dst = edge_index[1]

    W1a = W1[:D]
    W1b = W1[D:2 * D]
    w1r = W1[2 * D:2 * D + 1]
    W1e = W1[2 * D + 1:]
    W3a = W3[:D]
    W3b = W3[D:]

    xa, xb = _proj(x, W1a, W1b, b1.reshape(1, H))

    g = xa[dst] + xb[src]
    diff = pos[dst] - pos[src]
    r2 = jnp.sum(diff * diff, axis=-1, keepdims=True)

    m, gamma = _edge_mlp(g, r2, edge_attr, W1e, w1r, W2, b2.reshape(1, H),
                         W5, b5.reshape(1, 1))

    rinv = jax.lax.rsqrt(r2 + 1e-08)
    cvec = gamma * rinv * diff

    m_sum = jax.ops.segment_sum(m, dst, num_segments=N)
    deg = jax.ops.segment_sum(jnp.ones((E, 1), jnp.float32), dst, num_segments=N)
    coord = jax.ops.segment_sum(cvec, dst, num_segments=N)

    x_new, pos_new = _node_mlp(x, m_sum, deg, coord, pos, W3a, W3b,
                               b3.reshape(1, H), W4, b4.reshape(1, D))
    return (x_new, pos_new)


# speedup vs baseline: 2.0189x; 2.0189x over previous
import functools

import jax
import jax.numpy as jnp
from jax import lax
from jax.experimental import pallas as pl
from jax.experimental.pallas import tpu as pltpu
from jax.experimental.pallas import tpu_sc as plsc

N = 10000
E = 320000
D = 128
ED = 16
H = 128

EB = 1280
NB = 2000

NC = 2
NS = 16
NW = NC * NS
CH = 128
NCHUNK = 80
TPB = CH * NCHUNK
EP = NW * TPB
NPAD = 10112
NPS = NPAD // NS


def _silu(v):
    return v * (1.0 / (1.0 + jnp.exp(-v)))


def _proj_body(x_ref, w1a_ref, w1b_ref, b1_ref, xa_ref, xb_ref):
    x = x_ref[...]
    xa_ref[...] = jnp.dot(x, w1a_ref[...], preferred_element_type=jnp.float32) + b1_ref[...]
    xb_ref[...] = jnp.dot(x, w1b_ref[...], preferred_element_type=jnp.float32)


def _proj(x, W1a, W1b, b1):
    return pl.pallas_call(
        _proj_body,
        grid=(N // NB,),
        in_specs=[
            pl.BlockSpec((NB, D), lambda i: (i, 0)),
            pl.BlockSpec((D, H), lambda i: (0, 0)),
            pl.BlockSpec((D, H), lambda i: (0, 0)),
            pl.BlockSpec((1, H), lambda i: (0, 0)),
        ],
        out_specs=[
            pl.BlockSpec((NB, H), lambda i: (i, 0)),
            pl.BlockSpec((NB, H), lambda i: (i, 0)),
        ],
        out_shape=[
            jax.ShapeDtypeStruct((N, H), jnp.float32),
            jax.ShapeDtypeStruct((N, H), jnp.float32),
        ],
    )(x, W1a, W1b, b1)


def _sc_gather_body(xa, xb, src2d, dst2d, px, py, pz,
                    g_out, r2_out, dx_out, dy_out, dz_out,
                    sidx_v, didx_v, px_v, py_v, pz_v, g_v,
                    r2_v, dx_v, dy_v, dz_v):
    c = lax.axis_index("c")
    s = lax.axis_index("s")
    t = s * NC + c
    rowbase = t * NCHUNK
    ebase = t * TPB

    pltpu.sync_copy(src2d.at[pl.ds(rowbase, NCHUNK)], sidx_v)
    pltpu.sync_copy(dst2d.at[pl.ds(rowbase, NCHUNK)], didx_v)
    pltpu.sync_copy(px, px_v)
    pltpu.sync_copy(py, py_v)
    pltpu.sync_copy(pz, pz_v)

    def chunk(j, carry):
        pltpu.sync_copy(xa.at[didx_v.at[j]], g_v)
        pltpu.sync_copy(xb.at[sidx_v.at[j]], g_v, add=True)
        pltpu.sync_copy(g_v, g_out.at[pl.ds(ebase + j * CH, CH)])
        for k in range(CH // 16):
            off = j * CH + k * 16
            di = didx_v[j, pl.ds(k * 16, 16)]
            si = sidx_v[j, pl.ds(k * 16, 16)]
            dx = plsc.load_gather(px_v, [di]) - plsc.load_gather(px_v, [si])
            dy = plsc.load_gather(py_v, [di]) - plsc.load_gather(py_v, [si])
            dz = plsc.load_gather(pz_v, [di]) - plsc.load_gather(pz_v, [si])
            r2_v[pl.ds(off, 16)] = dx * dx + dy * dy + dz * dz
            dx_v[pl.ds(off, 16)] = dx
            dy_v[pl.ds(off, 16)] = dy
            dz_v[pl.ds(off, 16)] = dz
        return carry

    lax.fori_loop(0, NCHUNK, chunk, 0)

    pltpu.sync_copy(r2_v, r2_out.at[pl.ds(ebase, TPB)])
    pltpu.sync_copy(dx_v, dx_out.at[pl.ds(ebase, TPB)])
    pltpu.sync_copy(dy_v, dy_out.at[pl.ds(ebase, TPB)])
    pltpu.sync_copy(dz_v, dz_out.at[pl.ds(ebase, TPB)])


def _sc_gather(xa, xb, src2d, dst2d, px, py, pz):
    mesh = plsc.VectorSubcoreMesh(core_axis_name="c", subcore_axis_name="s")
    f = pl.kernel(
        _sc_gather_body,
        out_type=[
            jax.ShapeDtypeStruct((EP, H), jnp.float32),
            jax.ShapeDtypeStruct((EP,), jnp.float32),
            jax.ShapeDtypeStruct((EP,), jnp.float32),
            jax.ShapeDtypeStruct((EP,), jnp.float32),
            jax.ShapeDtypeStruct((EP,), jnp.float32),
        ],
        mesh=mesh,
        scratch_types=[
            pltpu.VMEM((NCHUNK, CH), jnp.int32),
            pltpu.VMEM((NCHUNK, CH), jnp.int32),
            pltpu.VMEM((N,), jnp.float32),
            pltpu.VMEM((N,), jnp.float32),
            pltpu.VMEM((N,), jnp.float32),
            pltpu.VMEM((CH, H), jnp.float32),
            pltpu.VMEM((TPB,), jnp.float32),
            pltpu.VMEM((TPB,), jnp.float32),
            pltpu.VMEM((TPB,), jnp.float32),
            pltpu.VMEM((TPB,), jnp.float32),
        ],
        compiler_params=pltpu.CompilerParams(use_tc_tiling_on_sc=False, needs_layout_passes=False),
    )
    return f(xa, xb, src2d, dst2d, px, py, pz)


def _edge_body(g_ref, r2_ref, dx_ref, dy_ref, dz_ref, ea_ref, w1e_ref,
               w1r_ref, w2_ref, b2_ref, w5_ref, b5_ref, m_ref, cv_ref):
    r2 = r2_ref[...]
    pre1 = (g_ref[...]
            + r2 * w1r_ref[...]
            + jnp.dot(ea_ref[...], w1e_ref[...], preferred_element_type=jnp.float32))
    z1 = _silu(pre1)
    m = _silu(jnp.dot(z1, w2_ref[...], preferred_element_type=jnp.float32) + b2_ref[...])
    m_ref[...] = m
    gamma = jnp.dot(m, w5_ref[...], preferred_element_type=jnp.float32) + b5_ref[...]
    sc = gamma * lax.rsqrt(r2 + 1e-08)
    ones = jnp.ones_like(sc)
    cv_ref[...] = jnp.concatenate(
        [ones, sc * dx_ref[...], sc * dy_ref[...], sc * dz_ref[...]], axis=1)


def _edge_mlp(g, r2, dx, dy, dz, ea, W1e, w1r, W2, b2, W5, b5):
    return pl.pallas_call(
        _edge_body,
        grid=(EP // EB,),
        in_specs=[
            pl.BlockSpec((EB, H), lambda i: (i, 0)),
            pl.BlockSpec((EB, 1), lambda i: (i, 0)),
            pl.BlockSpec((EB, 1), lambda i: (i, 0)),
            pl.BlockSpec((EB, 1), lambda i: (i, 0)),
            pl.BlockSpec((EB, 1), lambda i: (i, 0)),
            pl.BlockSpec((EB, ED), lambda i: (i, 0)),
            pl.BlockSpec((ED, H), lambda i: (0, 0)),
            pl.BlockSpec((1, H), lambda i: (0, 0)),
            pl.BlockSpec((H, H), lambda i: (0, 0)),
            pl.BlockSpec((1, H), lambda i: (0, 0)),
            pl.BlockSpec((H, 1), lambda i: (0, 0)),
            pl.BlockSpec((1, 1), lambda i: (0, 0)),
        ],
        out_specs=[
            pl.BlockSpec((EB, H), lambda i: (i, 0)),
            pl.BlockSpec((EB, 4), lambda i: (i, 0)),
        ],
        out_shape=[
            jax.ShapeDtypeStruct((EP, H), jnp.float32),
            jax.ShapeDtypeStruct((EP, 4), jnp.float32),
        ],
        compiler_params=pltpu.CompilerParams(
            dimension_semantics=("arbitrary",),
        ),
    )(g, r2, dx, dy, dz, ea, W1e, w1r, W2, b2, W5, b5)


def _sc_scatter_body(m, dst2d, z2d, macc,
                     didx_v, m_v, m_sh):
    c = lax.axis_index("c")
    s = lax.axis_index("s")
    t = s * NC + c
    rowbase = t * NCHUNK
    ebase = t * TPB

    pltpu.sync_copy(z2d.at[pl.ds(s * NPS, NPS)], m_sh.at[pl.ds(s * NPS, NPS)])
    pltpu.sync_copy(dst2d.at[pl.ds(rowbase, NCHUNK)], didx_v)
    plsc.subcore_barrier()

    def chunk(j, carry):
        pltpu.sync_copy(m.at[pl.ds(ebase + j * CH, CH)], m_v)
        pltpu.sync_copy(m_v, m_sh.at[didx_v.at[j]], add=True)
        return carry

    lax.fori_loop(0, NCHUNK, chunk, 0)
    plsc.subcore_barrier()

    pltpu.sync_copy(m_sh.at[pl.ds(s * NPS, NPS)], macc.at[c, pl.ds(s * NPS, NPS)])


def _sc_scatter(m, dst2d, z2d):
    mesh = plsc.VectorSubcoreMesh(core_axis_name="c", subcore_axis_name="s")
    f = pl.kernel(
        _sc_scatter_body,
        out_type=[
            jax.ShapeDtypeStruct((NC, NPAD, H), jnp.float32),
        ],
        mesh=mesh,
        scratch_types=[
            pltpu.VMEM((NCHUNK, CH), jnp.int32),
            pltpu.VMEM((CH, H), jnp.float32),
            pltpu.VMEM_SHARED((NPAD, H), jnp.float32),
        ],
        compiler_params=pltpu.CompilerParams(use_tc_tiling_on_sc=False, needs_layout_passes=False),
    )
    return f(m, dst2d, z2d)


def _node_body(x_ref, ms_ref, deg_ref, coord_ref, pos_ref, w3a_ref, w3b_ref,
               b3_ref, w4_ref, b4_ref, xn_ref, pn_ref):
    inv = 1.0 / jnp.maximum(deg_ref[...], 1.0)
    ms = ms_ref[...] * inv
    pre = (jnp.dot(x_ref[...], w3a_ref[...], preferred_element_type=jnp.float32)
           + jnp.dot(ms, w3b_ref[...], preferred_element_type=jnp.float32)
           + b3_ref[...])
    xn_ref[...] = jnp.dot(_silu(pre), w4_ref[...], preferred_element_type=jnp.float32) + b4_ref[...]
    pn_ref[...] = pos_ref[...] + coord_ref[...] * inv


def _node_mlp(x, m_sum, deg, coord, pos, W3a, W3b, b3, W4, b4):
    return pl.pallas_call(
        _node_body,
        grid=(N // NB,),
        in_specs=[
            pl.BlockSpec((NB, D), lambda i: (i, 0)),
            pl.BlockSpec((NB, H), lambda i: (i, 0)),
            pl.BlockSpec((NB, 1), lambda i: (i, 0)),
            pl.BlockSpec((NB, 3), lambda i: (i, 0)),
            pl.BlockSpec((NB, 3), lambda i: (i, 0)),
            pl.BlockSpec((D, H), lambda i: (0, 0)),
            pl.BlockSpec((H, H), lambda i: (0, 0)),
            pl.BlockSpec((1, H), lambda i: (0, 0)),
            pl.BlockSpec((H, D), lambda i: (0, 0)),
            pl.BlockSpec((1, D), lambda i: (0, 0)),
        ],
        out_specs=[
            pl.BlockSpec((NB, D), lambda i: (i, 0)),
            pl.BlockSpec((NB, 3), lambda i: (i, 0)),
        ],
        out_shape=[
            jax.ShapeDtypeStruct((N, D), jnp.float32),
            jax.ShapeDtypeStruct((N, 3), jnp.float32),
        ],
    )(x, m_sum, deg, coord, pos, W3a, W3b, b3, W4, b4)


def kernel(x, pos, edge_index, edge_attr, W1, b1, W2, b2, W3, b3, W4, b4, W5, b5):
    src = edge_index[0]
    dst = edge_index[1]

    W1a = W1[:D]
    W1b = W1[D:2 * D]
    w1r = W1[2 * D:2 * D + 1]
    W1e = W1[2 * D + 1:]
    W3a = W3[:D]
    W3b = W3[D:]

    xa, xb = _proj(x, W1a, W1b, b1.reshape(1, H))

    pad = EP - E
    zpad_i = jnp.zeros((pad,), jnp.int32)
    src_g2d = jnp.concatenate([src, zpad_i]).reshape(EP // CH, CH)
    dst_g2d = jnp.concatenate([dst, zpad_i]).reshape(EP // CH, CH)
    dst_s2d = jnp.concatenate([dst, jnp.full((pad,), N, jnp.int32)]).reshape(EP // CH, CH)
    ea_pad = jnp.concatenate([edge_attr, jnp.zeros((pad, ED), jnp.float32)])

    px = pos[:, 0]
    py = pos[:, 1]
    pz = pos[:, 2]

    g, r2f, dxf, dyf, dzf = _sc_gather(xa, xb, src_g2d, dst_g2d, px, py, pz)

    m, cvec = _edge_mlp(g, r2f.reshape(EP, 1), dxf.reshape(EP, 1),
                        dyf.reshape(EP, 1), dzf.reshape(EP, 1), ea_pad,
                        W1e, w1r, W2, b2.reshape(1, H), W5, b5.reshape(1, 1))

    macc = _sc_scatter(m, dst_s2d, jnp.zeros((NPAD, H), jnp.float32))[0]
    m_sum = macc[0, :N] + macc[1, :N]

    dst_p = dst_s2d.reshape(EP)
    cs = jax.ops.segment_sum(cvec, dst_p, num_segments=NPAD)[:N]
    deg = cs[:, :1]
    coord = cs[:, 1:4]

    x_new, pos_new = _node_mlp(x, m_sum, deg, coord, pos, W3a, W3b,
                               b3.reshape(1, H), W4, b4.reshape(1, D))
    return (x_new, pos_new)


# speedup vs baseline: 2.3998x; 1.1887x over previous
import functools

import jax
import jax.numpy as jnp
from jax import lax
from jax.experimental import pallas as pl
from jax.experimental.pallas import tpu as pltpu
from jax.experimental.pallas import tpu_sc as plsc

N = 10000
E = 320000
D = 128
ED = 16
H = 128

EB = 1280
NB = 2000

NC = 2
NS = 16
NW = NC * NS
CH = 128
NCHUNK = 80
TPB = CH * NCHUNK
EP = NW * TPB
NPAD = 10112
NPS = NPAD // NS


def _silu(v):
    return v * (1.0 / (1.0 + jnp.exp(-v)))


def _proj_body(x_ref, w1a_ref, w1b_ref, b1_ref, xa_ref, xb_ref):
    x = x_ref[...]
    xa_ref[...] = jnp.dot(x, w1a_ref[...], preferred_element_type=jnp.float32) + b1_ref[...]
    xb_ref[...] = jnp.dot(x, w1b_ref[...], preferred_element_type=jnp.float32)


def _proj(x, W1a, W1b, b1):
    return pl.pallas_call(
        _proj_body,
        grid=(N // NB,),
        in_specs=[
            pl.BlockSpec((NB, D), lambda i: (i, 0)),
            pl.BlockSpec((D, H), lambda i: (0, 0)),
            pl.BlockSpec((D, H), lambda i: (0, 0)),
            pl.BlockSpec((1, H), lambda i: (0, 0)),
        ],
        out_specs=[
            pl.BlockSpec((NB, H), lambda i: (i, 0)),
            pl.BlockSpec((NB, H), lambda i: (i, 0)),
        ],
        out_shape=[
            jax.ShapeDtypeStruct((N, H), jnp.float32),
            jax.ShapeDtypeStruct((N, H), jnp.float32),
        ],
    )(x, W1a, W1b, b1)


def _sc_gather_body(xa, xb, src2d, dst2d, px, py, pz,
                    g_out, r2_out, dx_out, dy_out, dz_out,
                    sidx_v, didx_v, px_v, py_v, pz_v, g_v,
                    r2_v, dx_v, dy_v, dz_v):
    c = lax.axis_index("c")
    s = lax.axis_index("s")
    t = s * NC + c
    rowbase = t * NCHUNK
    ebase = t * TPB

    pltpu.sync_copy(src2d.at[pl.ds(rowbase, NCHUNK)], sidx_v)
    pltpu.sync_copy(dst2d.at[pl.ds(rowbase, NCHUNK)], didx_v)
    pltpu.sync_copy(px, px_v)
    pltpu.sync_copy(py, py_v)
    pltpu.sync_copy(pz, pz_v)

    def chunk(j, carry):
        pltpu.sync_copy(xa.at[didx_v.at[j]], g_v)
        pltpu.sync_copy(xb.at[sidx_v.at[j]], g_v, add=True)
        pltpu.sync_copy(g_v, g_out.at[pl.ds(ebase + j * CH, CH)])
        for k in range(CH // 16):
            off = j * CH + k * 16
            di = didx_v[j, pl.ds(k * 16, 16)]
            si = sidx_v[j, pl.ds(k * 16, 16)]
            dx = plsc.load_gather(px_v, [di]) - plsc.load_gather(px_v, [si])
            dy = plsc.load_gather(py_v, [di]) - plsc.load_gather(py_v, [si])
            dz = plsc.load_gather(pz_v, [di]) - plsc.load_gather(pz_v, [si])
            r2_v[pl.ds(off, 16)] = dx * dx + dy * dy + dz * dz
            dx_v[pl.ds(off, 16)] = dx
            dy_v[pl.ds(off, 16)] = dy
            dz_v[pl.ds(off, 16)] = dz
        return carry

    lax.fori_loop(0, NCHUNK, chunk, 0)

    pltpu.sync_copy(r2_v, r2_out.at[pl.ds(ebase, TPB)])
    pltpu.sync_copy(dx_v, dx_out.at[pl.ds(ebase, TPB)])
    pltpu.sync_copy(dy_v, dy_out.at[pl.ds(ebase, TPB)])
    pltpu.sync_copy(dz_v, dz_out.at[pl.ds(ebase, TPB)])


def _sc_gather(xa, xb, src2d, dst2d, px, py, pz):
    mesh = plsc.VectorSubcoreMesh(core_axis_name="c", subcore_axis_name="s")
    f = pl.kernel(
        _sc_gather_body,
        out_type=[
            jax.ShapeDtypeStruct((EP, H), jnp.float32),
            jax.ShapeDtypeStruct((EP,), jnp.float32),
            jax.ShapeDtypeStruct((EP,), jnp.float32),
            jax.ShapeDtypeStruct((EP,), jnp.float32),
            jax.ShapeDtypeStruct((EP,), jnp.float32),
        ],
        mesh=mesh,
        scratch_types=[
            pltpu.VMEM((NCHUNK, CH), jnp.int32),
            pltpu.VMEM((NCHUNK, CH), jnp.int32),
            pltpu.VMEM((N,), jnp.float32),
            pltpu.VMEM((N,), jnp.float32),
            pltpu.VMEM((N,), jnp.float32),
            pltpu.VMEM((CH, H), jnp.float32),
            pltpu.VMEM((TPB,), jnp.float32),
            pltpu.VMEM((TPB,), jnp.float32),
            pltpu.VMEM((TPB,), jnp.float32),
            pltpu.VMEM((TPB,), jnp.float32),
        ],
        compiler_params=pltpu.CompilerParams(use_tc_tiling_on_sc=False, needs_layout_passes=False),
    )
    return f(xa, xb, src2d, dst2d, px, py, pz)


def _edge_body(g_ref, r2_ref, dx_ref, dy_ref, dz_ref, ea_ref, w1e_ref,
               w1r_ref, w2_ref, b2_ref, w5_ref, b5_ref, m_ref, cv_ref):
    r2 = r2_ref[...]
    pre1 = (g_ref[...]
            + r2 * w1r_ref[...]
            + jnp.dot(ea_ref[...], w1e_ref[...], preferred_element_type=jnp.float32))
    z1 = _silu(pre1)
    m = _silu(jnp.dot(z1, w2_ref[...], preferred_element_type=jnp.float32) + b2_ref[...])
    m_ref[...] = m
    gamma = jnp.dot(m, w5_ref[...], preferred_element_type=jnp.float32) + b5_ref[...]
    sc = gamma * lax.rsqrt(r2 + 1e-08)
    ones = jnp.ones_like(sc)
    cv_ref[...] = jnp.concatenate(
        [ones, sc * dx_ref[...], sc * dy_ref[...], sc * dz_ref[...]], axis=1)


def _edge_mlp(g, r2, dx, dy, dz, ea, W1e, w1r, W2, b2, W5, b5):
    return pl.pallas_call(
        _edge_body,
        grid=(EP // EB,),
        in_specs=[
            pl.BlockSpec((EB, H), lambda i: (i, 0)),
            pl.BlockSpec((EB, 1), lambda i: (i, 0)),
            pl.BlockSpec((EB, 1), lambda i: (i, 0)),
            pl.BlockSpec((EB, 1), lambda i: (i, 0)),
            pl.BlockSpec((EB, 1), lambda i: (i, 0)),
            pl.BlockSpec((EB, ED), lambda i: (i, 0)),
            pl.BlockSpec((ED, H), lambda i: (0, 0)),
            pl.BlockSpec((1, H), lambda i: (0, 0)),
            pl.BlockSpec((H, H), lambda i: (0, 0)),
            pl.BlockSpec((1, H), lambda i: (0, 0)),
            pl.BlockSpec((H, 1), lambda i: (0, 0)),
            pl.BlockSpec((1, 1), lambda i: (0, 0)),
        ],
        out_specs=[
            pl.BlockSpec((EB, H), lambda i: (i, 0)),
            pl.BlockSpec((EB, 4), lambda i: (i, 0)),
        ],
        out_shape=[
            jax.ShapeDtypeStruct((EP, H), jnp.float32),
            jax.ShapeDtypeStruct((EP, 4), jnp.float32),
        ],
        compiler_params=pltpu.CompilerParams(
            dimension_semantics=("arbitrary",),
        ),
    )(g, r2, dx, dy, dz, ea, W1e, w1r, W2, b2, W5, b5)


CV2D = EP * 4 // 128
NP4 = NPAD * 4


def _sc_scatter_body(m, dst2d, z2d, macc,
                     didx_v, m_v, m_sh):
    c = lax.axis_index("c")
    s = lax.axis_index("s")
    t = s * NC + c
    rowbase = t * NCHUNK
    ebase = t * TPB

    pltpu.sync_copy(z2d.at[pl.ds(s * NPS, NPS)], m_sh.at[pl.ds(s * NPS, NPS)])
    pltpu.sync_copy(dst2d.at[pl.ds(rowbase, NCHUNK)], didx_v)
    plsc.subcore_barrier()

    def chunk(j, carry):
        pltpu.sync_copy(m.at[pl.ds(ebase + j * CH, CH)], m_v)
        pltpu.sync_copy(m_v, m_sh.at[didx_v.at[j]], add=True)
        return carry

    lax.fori_loop(0, NCHUNK, chunk, 0)
    plsc.subcore_barrier()

    pltpu.sync_copy(m_sh.at[pl.ds(s * NPS, NPS)], macc.at[c, pl.ds(s * NPS, NPS)])


def _sc_scatter(m, dst2d, z2d):
    mesh = plsc.VectorSubcoreMesh(core_axis_name="c", subcore_axis_name="s")
    f = pl.kernel(
        _sc_scatter_body,
        out_type=[
            jax.ShapeDtypeStruct((NC, NPAD, H), jnp.float32),
        ],
        mesh=mesh,
        scratch_types=[
            pltpu.VMEM((NCHUNK, CH), jnp.int32),
            pltpu.VMEM((CH, H), jnp.float32),
            pltpu.VMEM_SHARED((NPAD, H), jnp.float32),
        ],
        compiler_params=pltpu.CompilerParams(use_tc_tiling_on_sc=False, needs_layout_passes=False),
    )
    return f(m, dst2d, z2d)


def _sc_cpath_body(cvec2d, dst2d, z1, cw, didx_v, c_v, cacc_v):
    c = lax.axis_index("c")
    s = lax.axis_index("s")
    t = s * NC + c
    rowbase = t * NCHUNK
    ebase = t * TPB

    pltpu.sync_copy(z1, cacc_v)
    pltpu.sync_copy(dst2d.at[pl.ds(rowbase, NCHUNK)], didx_v)

    iota16 = lax.iota(jnp.int32, 16)

    def chunk(j, carry):
        pltpu.sync_copy(cvec2d.at[pl.ds((ebase + j * CH) * 4 // 128, CH * 4 // 128)], c_v)
        for e0 in range(0, CH, 16):
            didx16 = didx_v[j, pl.ds(e0, 16)]
            addr = didx16 * 4
            for k in range(4):
                fl = iota16 * 4 + (e0 * 4 + k)
                vals = plsc.load_gather(c_v, [fl >> 7, fl & 127])
                plsc.addupdate_scatter(cacc_v, [addr + k], vals)
        return carry

    lax.fori_loop(0, NCHUNK, chunk, 0)
    pltpu.sync_copy(cacc_v, cw.at[t])


def _sc_cpath(cvec2d, dst2d, z1):
    mesh = plsc.VectorSubcoreMesh(core_axis_name="c", subcore_axis_name="s")
    f = pl.kernel(
        _sc_cpath_body,
        out_type=[
            jax.ShapeDtypeStruct((NW, NP4), jnp.float32),
        ],
        mesh=mesh,
        scratch_types=[
            pltpu.VMEM((NCHUNK, CH), jnp.int32),
            pltpu.VMEM((CH * 4 // 128, 128), jnp.float32),
            pltpu.VMEM((NP4,), jnp.float32),
        ],
        compiler_params=pltpu.CompilerParams(use_tc_tiling_on_sc=False, needs_layout_passes=False),
    )
    return f(cvec2d, dst2d, z1)


def _node_body(x_ref, ms_ref, deg_ref, coord_ref, pos_ref, w3a_ref, w3b_ref,
               b3_ref, w4_ref, b4_ref, xn_ref, pn_ref):
    inv = 1.0 / jnp.maximum(deg_ref[...], 1.0)
    ms = ms_ref[...] * inv
    pre = (jnp.dot(x_ref[...], w3a_ref[...], preferred_element_type=jnp.float32)
           + jnp.dot(ms, w3b_ref[...], preferred_element_type=jnp.float32)
           + b3_ref[...])
    xn_ref[...] = jnp.dot(_silu(pre), w4_ref[...], preferred_element_type=jnp.float32) + b4_ref[...]
    pn_ref[...] = pos_ref[...] + coord_ref[...] * inv


def _node_mlp(x, m_sum, deg, coord, pos, W3a, W3b, b3, W4, b4):
    return pl.pallas_call(
        _node_body,
        grid=(N // NB,),
        in_specs=[
            pl.BlockSpec((NB, D), lambda i: (i, 0)),
            pl.BlockSpec((NB, H), lambda i: (i, 0)),
            pl.BlockSpec((NB, 1), lambda i: (i, 0)),
            pl.BlockSpec((NB, 3), lambda i: (i, 0)),
            pl.BlockSpec((NB, 3), lambda i: (i, 0)),
            pl.BlockSpec((D, H), lambda i: (0, 0)),
            pl.BlockSpec((H, H), lambda i: (0, 0)),
            pl.BlockSpec((1, H), lambda i: (0, 0)),
            pl.BlockSpec((H, D), lambda i: (0, 0)),
            pl.BlockSpec((1, D), lambda i: (0, 0)),
        ],
        out_specs=[
            pl.BlockSpec((NB, D), lambda i: (i, 0)),
            pl.BlockSpec((NB, 3), lambda i: (i, 0)),
        ],
        out_shape=[
            jax.ShapeDtypeStruct((N, D), jnp.float32),
            jax.ShapeDtypeStruct((N, 3), jnp.float32),
        ],
    )(x, m_sum, deg, coord, pos, W3a, W3b, b3, W4, b4)


def kernel(x, pos, edge_index, edge_attr, W1, b1, W2, b2, W3, b3, W4, b4, W5, b5):
    src = edge_index[0]
    dst = edge_index[1]

    W1a = W1[:D]
    W1b = W1[D:2 * D]
    w1r = W1[2 * D:2 * D + 1]
    W1e = W1[2 * D + 1:]
    W3a = W3[:D]
    W3b = W3[D:]

    xa, xb = _proj(x, W1a, W1b, b1.reshape(1, H))

    pad = EP - E
    zpad_i = jnp.zeros((pad,), jnp.int32)
    src_g2d = jnp.concatenate([src, zpad_i]).reshape(EP // CH, CH)
    dst_g2d = jnp.concatenate([dst, zpad_i]).reshape(EP // CH, CH)
    dst_s2d = jnp.concatenate([dst, jnp.full((pad,), N, jnp.int32)]).reshape(EP // CH, CH)
    ea_pad = jnp.concatenate([edge_attr, jnp.zeros((pad, ED), jnp.float32)])

    px = pos[:, 0]
    py = pos[:, 1]
    pz = pos[:, 2]

    g, r2f, dxf, dyf, dzf = _sc_gather(xa, xb, src_g2d, dst_g2d, px, py, pz)

    m, cvec = _edge_mlp(g, r2f.reshape(EP, 1), dxf.reshape(EP, 1),
                        dyf.reshape(EP, 1), dzf.reshape(EP, 1), ea_pad,
                        W1e, w1r, W2, b2.reshape(1, H), W5, b5.reshape(1, 1))

    macc = _sc_scatter(m, dst_s2d, jnp.zeros((NPAD, H), jnp.float32))[0]
    cw = _sc_cpath(cvec.reshape(CV2D, 128), dst_s2d, jnp.zeros((NP4,), jnp.float32))[0]
    m_sum = macc[0, :N] + macc[1, :N]
    cs = cw.sum(axis=0).reshape(NPAD, 4)[:N]
    deg = cs[:, :1]
    coord = cs[:, 1:4]

    x_new, pos_new = _node_mlp(x, m_sum, deg, coord, pos, W3a, W3b,
                               b3.reshape(1, H), W4, b4.reshape(1, D))
    return (x_new, pos_new)


# speedup vs baseline: 2.6664x; 1.1111x over previous
import functools

import jax
import jax.numpy as jnp
from jax import lax
from jax.experimental import pallas as pl
from jax.experimental.pallas import tpu as pltpu
from jax.experimental.pallas import tpu_sc as plsc

N = 10000
E = 320000
D = 128
ED = 16
H = 128

EB = 1280
NB = 2000

NC = 2
NS = 16
NW = NC * NS
CH = 128
NCHUNK = 80
TPB = CH * NCHUNK
EP = NW * TPB
NPAD = 10112
NPS = NPAD // NS


def _silu(v):
    return v * (1.0 / (1.0 + jnp.exp(-v)))


def _proj_body(x_ref, w1a_ref, w1b_ref, b1_ref, xa_ref, xb_ref):
    x = x_ref[...]
    xa_ref[...] = jnp.dot(x, w1a_ref[...], preferred_element_type=jnp.float32) + b1_ref[...]
    xb_ref[...] = jnp.dot(x, w1b_ref[...], preferred_element_type=jnp.float32)


def _proj(x, W1a, W1b, b1):
    return pl.pallas_call(
        _proj_body,
        grid=(N // NB,),
        in_specs=[
            pl.BlockSpec((NB, D), lambda i: (i, 0)),
            pl.BlockSpec((D, H), lambda i: (0, 0)),
            pl.BlockSpec((D, H), lambda i: (0, 0)),
            pl.BlockSpec((1, H), lambda i: (0, 0)),
        ],
        out_specs=[
            pl.BlockSpec((NB, H), lambda i: (i, 0)),
            pl.BlockSpec((NB, H), lambda i: (i, 0)),
        ],
        out_shape=[
            jax.ShapeDtypeStruct((N, H), jnp.float32),
            jax.ShapeDtypeStruct((N, H), jnp.float32),
        ],
    )(x, W1a, W1b, b1)


NITER = NCHUNK // 2


def _sc_gather_body(xa, xb, src2d, dst2d, px, py, pz,
                    g_out, r2_out, dx_out, dy_out, dz_out,
                    sidx_v, didx_v, px_v, py_v, pz_v, g_v0, g_v1,
                    r2_v, dx_v, dy_v, dz_v,
                    sem_a0, sem_a1, sem_b0, sem_b1, sem_w0, sem_w1):
    c = lax.axis_index("c")
    s = lax.axis_index("s")
    t = s * NC + c
    rowbase = t * NCHUNK
    ebase = t * TPB

    pltpu.sync_copy(src2d.at[pl.ds(rowbase, NCHUNK)], sidx_v)
    pltpu.sync_copy(dst2d.at[pl.ds(rowbase, NCHUNK)], didx_v)
    pltpu.sync_copy(px, px_v)
    pltpu.sync_copy(py, py_v)
    pltpu.sync_copy(pz, pz_v)

    def geometry(j):
        for k in range(CH // 16):
            off = j * CH + k * 16
            di = didx_v[j, pl.ds(k * 16, 16)]
            si = sidx_v[j, pl.ds(k * 16, 16)]
            dx = plsc.load_gather(px_v, [di]) - plsc.load_gather(px_v, [si])
            dy = plsc.load_gather(py_v, [di]) - plsc.load_gather(py_v, [si])
            dz = plsc.load_gather(pz_v, [di]) - plsc.load_gather(pz_v, [si])
            r2_v[pl.ds(off, 16)] = dx * dx + dy * dy + dz * dz
            dx_v[pl.ds(off, 16)] = dx
            dy_v[pl.ds(off, 16)] = dy
            dz_v[pl.ds(off, 16)] = dz

    def wait(src_ref, dst_ref, sem):
        pltpu.make_async_copy(src_ref, dst_ref, sem).wait()

    pltpu.async_copy(xa.at[didx_v.at[0]], g_v0, sem_a0)

    def body(jj, carry):
        a = 2 * jj
        b = 2 * jj + 1
        wait(xa.at[didx_v.at[a]], g_v0, sem_a0)
        pltpu.async_copy(xb.at[sidx_v.at[a]], g_v0, sem_b0, add=True)

        @pl.when(jj > 0)
        def _():
            wait(g_v1, g_out.at[pl.ds(ebase + (a - 1) * CH, CH)], sem_w1)

        pltpu.async_copy(xa.at[didx_v.at[b]], g_v1, sem_a1)

        geometry(a)

        wait(xb.at[sidx_v.at[a]], g_v0, sem_b0)
        pltpu.async_copy(g_v0, g_out.at[pl.ds(ebase + a * CH, CH)], sem_w0)

        wait(xa.at[didx_v.at[b]], g_v1, sem_a1)
        pltpu.async_copy(xb.at[sidx_v.at[b]], g_v1, sem_b1, add=True)

        geometry(b)

        wait(xb.at[sidx_v.at[b]], g_v1, sem_b1)
        pltpu.async_copy(g_v1, g_out.at[pl.ds(ebase + b * CH, CH)], sem_w1)

        @pl.when(jj + 1 < NITER)
        def _():
            wait(g_v0, g_out.at[pl.ds(ebase + a * CH, CH)], sem_w0)
            pltpu.async_copy(xa.at[didx_v.at[a + 2]], g_v0, sem_a0)

        return carry

    lax.fori_loop(0, NITER, body, 0)

    wait(g_v0, g_out.at[pl.ds(ebase, CH)], sem_w0)
    wait(g_v1, g_out.at[pl.ds(ebase, CH)], sem_w1)

    pltpu.sync_copy(r2_v, r2_out.at[pl.ds(ebase, TPB)])
    pltpu.sync_copy(dx_v, dx_out.at[pl.ds(ebase, TPB)])
    pltpu.sync_copy(dy_v, dy_out.at[pl.ds(ebase, TPB)])
    pltpu.sync_copy(dz_v, dz_out.at[pl.ds(ebase, TPB)])


def _sc_gather(xa, xb, src2d, dst2d, px, py, pz):
    mesh = plsc.VectorSubcoreMesh(core_axis_name="c", subcore_axis_name="s")
    f = pl.kernel(
        _sc_gather_body,
        out_type=[
            jax.ShapeDtypeStruct((EP, H), jnp.float32),
            jax.ShapeDtypeStruct((EP,), jnp.float32),
            jax.ShapeDtypeStruct((EP,), jnp.float32),
            jax.ShapeDtypeStruct((EP,), jnp.float32),
            jax.ShapeDtypeStruct((EP,), jnp.float32),
        ],
        mesh=mesh,
        scratch_types=[
            pltpu.VMEM((NCHUNK, CH), jnp.int32),
            pltpu.VMEM((NCHUNK, CH), jnp.int32),
            pltpu.VMEM((N,), jnp.float32),
            pltpu.VMEM((N,), jnp.float32),
            pltpu.VMEM((N,), jnp.float32),
            pltpu.VMEM((CH, H), jnp.float32),
            pltpu.VMEM((CH, H), jnp.float32),
            pltpu.VMEM((TPB,), jnp.float32),
            pltpu.VMEM((TPB,), jnp.float32),
            pltpu.VMEM((TPB,), jnp.float32),
            pltpu.VMEM((TPB,), jnp.float32),
            pltpu.SemaphoreType.DMA,
            pltpu.SemaphoreType.DMA,
            pltpu.SemaphoreType.DMA,
            pltpu.SemaphoreType.DMA,
            pltpu.SemaphoreType.DMA,
            pltpu.SemaphoreType.DMA,
        ],
        compiler_params=pltpu.CompilerParams(use_tc_tiling_on_sc=False, needs_layout_passes=False),
    )
    return f(xa, xb, src2d, dst2d, px, py, pz)


def _edge_body(g_ref, r2_ref, dx_ref, dy_ref, dz_ref, ea_ref, w1e_ref,
               w1r_ref, w2_ref, b2_ref, w5_ref, b5_ref, m_ref, cv_ref):
    r2 = r2_ref[...]
    pre1 = (g_ref[...]
            + r2 * w1r_ref[...]
            + jnp.dot(ea_ref[...], w1e_ref[...], preferred_element_type=jnp.float32))
    z1 = _silu(pre1)
    m = _silu(jnp.dot(z1, w2_ref[...], preferred_element_type=jnp.float32) + b2_ref[...])
    m_ref[...] = m
    gamma = jnp.dot(m, w5_ref[...], preferred_element_type=jnp.float32) + b5_ref[...]
    sc = gamma * lax.rsqrt(r2 + 1e-08)
    ones = jnp.ones_like(sc)
    cv_ref[...] = jnp.concatenate(
        [ones, sc * dx_ref[...], sc * dy_ref[...], sc * dz_ref[...]], axis=1)


def _edge_mlp(g, r2, dx, dy, dz, ea, W1e, w1r, W2, b2, W5, b5):
    return pl.pallas_call(
        _edge_body,
        grid=(EP // EB,),
        in_specs=[
            pl.BlockSpec((EB, H), lambda i: (i, 0)),
            pl.BlockSpec((EB, 1), lambda i: (i, 0)),
            pl.BlockSpec((EB, 1), lambda i: (i, 0)),
            pl.BlockSpec((EB, 1), lambda i: (i, 0)),
            pl.BlockSpec((EB, 1), lambda i: (i, 0)),
            pl.BlockSpec((EB, ED), lambda i: (i, 0)),
            pl.BlockSpec((ED, H), lambda i: (0, 0)),
            pl.BlockSpec((1, H), lambda i: (0, 0)),
            pl.BlockSpec((H, H), lambda i: (0, 0)),
            pl.BlockSpec((1, H), lambda i: (0, 0)),
            pl.BlockSpec((H, 1), lambda i: (0, 0)),
            pl.BlockSpec((1, 1), lambda i: (0, 0)),
        ],
        out_specs=[
            pl.BlockSpec((EB, H), lambda i: (i, 0)),
            pl.BlockSpec((EB, 4), lambda i: (i, 0)),
        ],
        out_shape=[
            jax.ShapeDtypeStruct((EP, H), jnp.float32),
            jax.ShapeDtypeStruct((EP, 4), jnp.float32),
        ],
        compiler_params=pltpu.CompilerParams(
            dimension_semantics=("arbitrary",),
        ),
    )(g, r2, dx, dy, dz, ea, W1e, w1r, W2, b2, W5, b5)


CV2D = EP * 4 // 128
NP4 = NPAD * 4


def _sc_scatter_body(m, dst2d, z2d, macc,
                     didx_v, m_v, m_sh):
    c = lax.axis_index("c")
    s = lax.axis_index("s")
    t = s * NC + c
    rowbase = t * NCHUNK
    ebase = t * TPB

    pltpu.sync_copy(z2d.at[pl.ds(s * NPS, NPS)], m_sh.at[pl.ds(s * NPS, NPS)])
    pltpu.sync_copy(dst2d.at[pl.ds(rowbase, NCHUNK)], didx_v)
    plsc.subcore_barrier()

    def chunk(j, carry):
        pltpu.sync_copy(m.at[pl.ds(ebase + j * CH, CH)], m_v)
        pltpu.sync_copy(m_v, m_sh.at[didx_v.at[j]], add=True)
        return carry

    lax.fori_loop(0, NCHUNK, chunk, 0)
    plsc.subcore_barrier()

    pltpu.sync_copy(m_sh.at[pl.ds(s * NPS, NPS)], macc.at[c, pl.ds(s * NPS, NPS)])


def _sc_scatter(m, dst2d, z2d):
    mesh = plsc.VectorSubcoreMesh(core_axis_name="c", subcore_axis_name="s")
    f = pl.kernel(
        _sc_scatter_body,
        out_type=[
            jax.ShapeDtypeStruct((NC, NPAD, H), jnp.float32),
        ],
        mesh=mesh,
        scratch_types=[
            pltpu.VMEM((NCHUNK, CH), jnp.int32),
            pltpu.VMEM((CH, H), jnp.float32),
            pltpu.VMEM_SHARED((NPAD, H), jnp.float32),
        ],
        compiler_params=pltpu.CompilerParams(use_tc_tiling_on_sc=False, needs_layout_passes=False),
    )
    return f(m, dst2d, z2d)


def _sc_cpath_body(cvec2d, dst2d, z1, cw, didx_v, c_v, cacc_v):
    c = lax.axis_index("c")
    s = lax.axis_index("s")
    t = s * NC + c
    rowbase = t * NCHUNK
    ebase = t * TPB

    pltpu.sync_copy(z1, cacc_v)
    pltpu.sync_copy(dst2d.at[pl.ds(rowbase, NCHUNK)], didx_v)

    iota16 = lax.iota(jnp.int32, 16)

    def chunk(j, carry):
        pltpu.sync_copy(cvec2d.at[pl.ds((ebase + j * CH) * 4 // 128, CH * 4 // 128)], c_v)
        for e0 in range(0, CH, 16):
            didx16 = didx_v[j, pl.ds(e0, 16)]
            addr = didx16 * 4
            for k in range(4):
                fl = iota16 * 4 + (e0 * 4 + k)
                vals = plsc.load_gather(c_v, [fl >> 7, fl & 127])
                plsc.addupdate_scatter(cacc_v, [addr + k], vals)
        return carry

    lax.fori_loop(0, NCHUNK, chunk, 0)
    pltpu.sync_copy(cacc_v, cw.at[t])


def _sc_cpath(cvec2d, dst2d, z1):
    mesh = plsc.VectorSubcoreMesh(core_axis_name="c", subcore_axis_name="s")
    f = pl.kernel(
        _sc_cpath_body,
        out_type=[
            jax.ShapeDtypeStruct((NW, NP4), jnp.float32),
        ],
        mesh=mesh,
        scratch_types=[
            pltpu.VMEM((NCHUNK, CH), jnp.int32),
            pltpu.VMEM((CH * 4 // 128, 128), jnp.float32),
            pltpu.VMEM((NP4,), jnp.float32),
        ],
        compiler_params=pltpu.CompilerParams(use_tc_tiling_on_sc=False, needs_layout_passes=False),
    )
    return f(cvec2d, dst2d, z1)


def _node_body(x_ref, ms_ref, deg_ref, coord_ref, pos_ref, w3a_ref, w3b_ref,
               b3_ref, w4_ref, b4_ref, xn_ref, pn_ref):
    inv = 1.0 / jnp.maximum(deg_ref[...], 1.0)
    ms = ms_ref[...] * inv
    pre = (jnp.dot(x_ref[...], w3a_ref[...], preferred_element_type=jnp.float32)
           + jnp.dot(ms, w3b_ref[...], preferred_element_type=jnp.float32)
           + b3_ref[...])
    xn_ref[...] = jnp.dot(_silu(pre), w4_ref[...], preferred_element_type=jnp.float32) + b4_ref[...]
    pn_ref[...] = pos_ref[...] + coord_ref[...] * inv


def _node_mlp(x, m_sum, deg, coord, pos, W3a, W3b, b3, W4, b4):
    return pl.pallas_call(
        _node_body,
        grid=(N // NB,),
        in_specs=[
            pl.BlockSpec((NB, D), lambda i: (i, 0)),
            pl.BlockSpec((NB, H), lambda i: (i, 0)),
            pl.BlockSpec((NB, 1), lambda i: (i, 0)),
            pl.BlockSpec((NB, 3), lambda i: (i, 0)),
            pl.BlockSpec((NB, 3), lambda i: (i, 0)),
            pl.BlockSpec((D, H), lambda i: (0, 0)),
            pl.BlockSpec((H, H), lambda i: (0, 0)),
            pl.BlockSpec((1, H), lambda i: (0, 0)),
            pl.BlockSpec((H, D), lambda i: (0, 0)),
            pl.BlockSpec((1, D), lambda i: (0, 0)),
        ],
        out_specs=[
            pl.BlockSpec((NB, D), lambda i: (i, 0)),
            pl.BlockSpec((NB, 3), lambda i: (i, 0)),
        ],
        out_shape=[
            jax.ShapeDtypeStruct((N, D), jnp.float32),
            jax.ShapeDtypeStruct((N, 3), jnp.float32),
        ],
    )(x, m_sum, deg, coord, pos, W3a, W3b, b3, W4, b4)


def kernel(x, pos, edge_index, edge_attr, W1, b1, W2, b2, W3, b3, W4, b4, W5, b5):
    src = edge_index[0]
    dst = edge_index[1]

    W1a = W1[:D]
    W1b = W1[D:2 * D]
    w1r = W1[2 * D:2 * D + 1]
    W1e = W1[2 * D + 1:]
    W3a = W3[:D]
    W3b = W3[D:]

    xa, xb = _proj(x, W1a, W1b, b1.reshape(1, H))

    pad = EP - E
    zpad_i = jnp.zeros((pad,), jnp.int32)
    src_g2d = jnp.concatenate([src, zpad_i]).reshape(EP // CH, CH)
    dst_g2d = jnp.concatenate([dst, zpad_i]).reshape(EP // CH, CH)
    dst_s2d = jnp.concatenate([dst, jnp.full((pad,), N, jnp.int32)]).reshape(EP // CH, CH)
    ea_pad = jnp.concatenate([edge_attr, jnp.zeros((pad, ED), jnp.float32)])

    px = pos[:, 0]
    py = pos[:, 1]
    pz = pos[:, 2]

    g, r2f, dxf, dyf, dzf = _sc_gather(xa, xb, src_g2d, dst_g2d, px, py, pz)

    m, cvec = _edge_mlp(g, r2f.reshape(EP, 1), dxf.reshape(EP, 1),
                        dyf.reshape(EP, 1), dzf.reshape(EP, 1), ea_pad,
                        W1e, w1r, W2, b2.reshape(1, H), W5, b5.reshape(1, 1))

    macc = _sc_scatter(m, dst_s2d, jnp.zeros((NPAD, H), jnp.float32))[0]
    cw = _sc_cpath(cvec.reshape(CV2D, 128), dst_s2d, jnp.zeros((NP4,), jnp.float32))[0]
    m_sum = macc[0, :N] + macc[1, :N]
    cs = cw.sum(axis=0).reshape(NPAD, 4)[:N]
    deg = cs[:, :1]
    coord = cs[:, 1:4]

    x_new, pos_new = _node_mlp(x, m_sum, deg, coord, pos, W3a, W3b,
                               b3.reshape(1, H), W4, b4.reshape(1, D))
    return (x_new, pos_new)


# speedup vs baseline: 2.9558x; 1.1085x over previous
import functools

import jax
import jax.numpy as jnp
from jax import lax
from jax.experimental import pallas as pl
from jax.experimental.pallas import tpu as pltpu
from jax.experimental.pallas import tpu_sc as plsc

N = 10000
E = 320000
D = 128
ED = 16
H = 128

EB = 1280
NB = 2000

NC = 2
NS = 16
NW = NC * NS
CH = 128
NCHUNK = 80
TPB = CH * NCHUNK
EP = NW * TPB
NPAD = 10112
NPS = NPAD // NS


def _silu(v):
    return v * (1.0 / (1.0 + jnp.exp(-v)))


def _proj_body(x_ref, w1a_ref, w1b_ref, b1_ref, xa_ref, xb_ref):
    x = x_ref[...]
    xa_ref[...] = jnp.dot(x, w1a_ref[...], preferred_element_type=jnp.float32) + b1_ref[...]
    xb_ref[...] = jnp.dot(x, w1b_ref[...], preferred_element_type=jnp.float32)


def _proj(x, W1a, W1b, b1):
    return pl.pallas_call(
        _proj_body,
        grid=(N // NB,),
        in_specs=[
            pl.BlockSpec((NB, D), lambda i: (i, 0)),
            pl.BlockSpec((D, H), lambda i: (0, 0)),
            pl.BlockSpec((D, H), lambda i: (0, 0)),
            pl.BlockSpec((1, H), lambda i: (0, 0)),
        ],
        out_specs=[
            pl.BlockSpec((NB, H), lambda i: (i, 0)),
            pl.BlockSpec((NB, H), lambda i: (i, 0)),
        ],
        out_shape=[
            jax.ShapeDtypeStruct((N, H), jnp.float32),
            jax.ShapeDtypeStruct((N, H), jnp.float32),
        ],
    )(x, W1a, W1b, b1)


def _make_gather_body(nchunk):
  niter = nchunk // 2
  tpb = nchunk * CH

  def _sc_gather_body(xa, xb, src2d, dst2d, px, py, pz,
                      g_out, r2_out, dx_out, dy_out, dz_out,
                      sidx_v, didx_v, px_v, py_v, pz_v, g_v0, g_v1,
                      r2_v, dx_v, dy_v, dz_v,
                      sem_a0, sem_a1, sem_b0, sem_b1, sem_w0, sem_w1):
    c = lax.axis_index("c")
    s = lax.axis_index("s")
    t = s * NC + c
    rowbase = t * nchunk
    ebase = t * tpb

    pltpu.sync_copy(src2d.at[pl.ds(rowbase, nchunk)], sidx_v)
    pltpu.sync_copy(dst2d.at[pl.ds(rowbase, nchunk)], didx_v)
    pltpu.sync_copy(px, px_v)
    pltpu.sync_copy(py, py_v)
    pltpu.sync_copy(pz, pz_v)

    def geometry(j):
        for k in range(CH // 16):
            off = j * CH + k * 16
            di = didx_v[j, pl.ds(k * 16, 16)]
            si = sidx_v[j, pl.ds(k * 16, 16)]
            dx = plsc.load_gather(px_v, [di]) - plsc.load_gather(px_v, [si])
            dy = plsc.load_gather(py_v, [di]) - plsc.load_gather(py_v, [si])
            dz = plsc.load_gather(pz_v, [di]) - plsc.load_gather(pz_v, [si])
            r2_v[pl.ds(off, 16)] = dx * dx + dy * dy + dz * dz
            dx_v[pl.ds(off, 16)] = dx
            dy_v[pl.ds(off, 16)] = dy
            dz_v[pl.ds(off, 16)] = dz

    def wait(src_ref, dst_ref, sem):
        pltpu.make_async_copy(src_ref, dst_ref, sem).wait()

    pltpu.async_copy(xa.at[didx_v.at[0]], g_v0, sem_a0)

    def body(jj, carry):
        a = 2 * jj
        b = 2 * jj + 1
        wait(xa.at[didx_v.at[a]], g_v0, sem_a0)
        pltpu.async_copy(xb.at[sidx_v.at[a]], g_v0, sem_b0, add=True)

        @pl.when(jj > 0)
        def _():
            wait(g_v1, g_out.at[pl.ds(ebase + (a - 1) * CH, CH)], sem_w1)

        pltpu.async_copy(xa.at[didx_v.at[b]], g_v1, sem_a1)

        geometry(a)

        wait(xb.at[sidx_v.at[a]], g_v0, sem_b0)
        pltpu.async_copy(g_v0, g_out.at[pl.ds(ebase + a * CH, CH)], sem_w0)

        wait(xa.at[didx_v.at[b]], g_v1, sem_a1)
        pltpu.async_copy(xb.at[sidx_v.at[b]], g_v1, sem_b1, add=True)

        geometry(b)

        wait(xb.at[sidx_v.at[b]], g_v1, sem_b1)
        pltpu.async_copy(g_v1, g_out.at[pl.ds(ebase + b * CH, CH)], sem_w1)

        @pl.when(jj + 1 < niter)
        def _():
            wait(g_v0, g_out.at[pl.ds(ebase + a * CH, CH)], sem_w0)
            pltpu.async_copy(xa.at[didx_v.at[a + 2]], g_v0, sem_a0)

        return carry

    lax.fori_loop(0, niter, body, 0)

    wait(g_v0, g_out.at[pl.ds(ebase, CH)], sem_w0)
    wait(g_v1, g_out.at[pl.ds(ebase, CH)], sem_w1)

    pltpu.sync_copy(r2_v, r2_out.at[pl.ds(ebase, tpb)])
    pltpu.sync_copy(dx_v, dx_out.at[pl.ds(ebase, tpb)])
    pltpu.sync_copy(dy_v, dy_out.at[pl.ds(ebase, tpb)])
    pltpu.sync_copy(dz_v, dz_out.at[pl.ds(ebase, tpb)])

  return _sc_gather_body


def _sc_gather(xa, xb, src2d, dst2d, px, py, pz, nchunk):
    ep = NW * nchunk * CH
    tpb = nchunk * CH
    mesh = plsc.VectorSubcoreMesh(core_axis_name="c", subcore_axis_name="s")
    f = pl.kernel(
        _make_gather_body(nchunk),
        out_type=[
            jax.ShapeDtypeStruct((ep, H), jnp.float32),
            jax.ShapeDtypeStruct((ep,), jnp.float32),
            jax.ShapeDtypeStruct((ep,), jnp.float32),
            jax.ShapeDtypeStruct((ep,), jnp.float32),
            jax.ShapeDtypeStruct((ep,), jnp.float32),
        ],
        mesh=mesh,
        scratch_types=[
            pltpu.VMEM((nchunk, CH), jnp.int32),
            pltpu.VMEM((nchunk, CH), jnp.int32),
            pltpu.VMEM((N,), jnp.float32),
            pltpu.VMEM((N,), jnp.float32),
            pltpu.VMEM((N,), jnp.float32),
            pltpu.VMEM((CH, H), jnp.float32),
            pltpu.VMEM((CH, H), jnp.float32),
            pltpu.VMEM((tpb,), jnp.float32),
            pltpu.VMEM((tpb,), jnp.float32),
            pltpu.VMEM((tpb,), jnp.float32),
            pltpu.VMEM((tpb,), jnp.float32),
            pltpu.SemaphoreType.DMA,
            pltpu.SemaphoreType.DMA,
            pltpu.SemaphoreType.DMA,
            pltpu.SemaphoreType.DMA,
            pltpu.SemaphoreType.DMA,
            pltpu.SemaphoreType.DMA,
        ],
        compiler_params=pltpu.CompilerParams(use_tc_tiling_on_sc=False, needs_layout_passes=False),
    )
    return f(xa, xb, src2d, dst2d, px, py, pz)


def _edge_body(g_ref, r2_ref, dx_ref, dy_ref, dz_ref, ea_ref, w1e_ref,
               w1r_ref, w2_ref, b2_ref, w5_ref, b5_ref, m_ref, cv_ref):
    r2 = r2_ref[...]
    pre1 = (g_ref[...]
            + r2 * w1r_ref[...]
            + jnp.dot(ea_ref[...], w1e_ref[...], preferred_element_type=jnp.float32))
    z1 = _silu(pre1)
    m = _silu(jnp.dot(z1, w2_ref[...], preferred_element_type=jnp.float32) + b2_ref[...])
    m_ref[...] = m
    gamma = jnp.dot(m, w5_ref[...], preferred_element_type=jnp.float32) + b5_ref[...]
    sc = gamma * lax.rsqrt(r2 + 1e-08)
    ones = jnp.ones_like(sc)
    cv_ref[...] = jnp.concatenate(
        [ones, sc * dx_ref[...], sc * dy_ref[...], sc * dz_ref[...]], axis=1)


def _edge_mlp(g, r2, dx, dy, dz, ea, W1e, w1r, W2, b2, W5, b5, ep):
    return pl.pallas_call(
        _edge_body,
        grid=(ep // EB,),
        in_specs=[
            pl.BlockSpec((EB, H), lambda i: (i, 0)),
            pl.BlockSpec((EB, 1), lambda i: (i, 0)),
            pl.BlockSpec((EB, 1), lambda i: (i, 0)),
            pl.BlockSpec((EB, 1), lambda i: (i, 0)),
            pl.BlockSpec((EB, 1), lambda i: (i, 0)),
            pl.BlockSpec((EB, ED), lambda i: (i, 0)),
            pl.BlockSpec((ED, H), lambda i: (0, 0)),
            pl.BlockSpec((1, H), lambda i: (0, 0)),
            pl.BlockSpec((H, H), lambda i: (0, 0)),
            pl.BlockSpec((1, H), lambda i: (0, 0)),
            pl.BlockSpec((H, 1), lambda i: (0, 0)),
            pl.BlockSpec((1, 1), lambda i: (0, 0)),
        ],
        out_specs=[
            pl.BlockSpec((EB, H), lambda i: (i, 0)),
            pl.BlockSpec((EB, 4), lambda i: (i, 0)),
        ],
        out_shape=[
            jax.ShapeDtypeStruct((ep, H), jnp.float32),
            jax.ShapeDtypeStruct((ep, 4), jnp.float32),
        ],
        compiler_params=pltpu.CompilerParams(
            dimension_semantics=("arbitrary",),
        ),
    )(g, r2, dx, dy, dz, ea, W1e, w1r, W2, b2, W5, b5)


CV2D = EP * 4 // 128
NP4 = NPAD * 4


def _make_scatter_body(nchunk):
  tpb = nchunk * CH

  def _sc_scatter_body(m, dst2d, z2d, macc,
                       didx_v, m_v, m_sh):
    c = lax.axis_index("c")
    s = lax.axis_index("s")
    t = s * NC + c
    rowbase = t * nchunk
    ebase = t * tpb

    pltpu.sync_copy(z2d.at[pl.ds(s * NPS, NPS)], m_sh.at[pl.ds(s * NPS, NPS)])
    pltpu.sync_copy(dst2d.at[pl.ds(rowbase, nchunk)], didx_v)
    plsc.subcore_barrier()

    def chunk(j, carry):
        pltpu.sync_copy(m.at[pl.ds(ebase + j * CH, CH)], m_v)
        pltpu.sync_copy(m_v, m_sh.at[didx_v.at[j]], add=True)
        return carry

    lax.fori_loop(0, nchunk, chunk, 0)
    plsc.subcore_barrier()

    pltpu.sync_copy(m_sh.at[pl.ds(s * NPS, NPS)], macc.at[c, pl.ds(s * NPS, NPS)])

  return _sc_scatter_body


def _sc_scatter(m, dst2d, z2d, nchunk):
    mesh = plsc.VectorSubcoreMesh(core_axis_name="c", subcore_axis_name="s")
    f = pl.kernel(
        _make_scatter_body(nchunk),
        out_type=[
            jax.ShapeDtypeStruct((NC, NPAD, H), jnp.float32),
        ],
        mesh=mesh,
        scratch_types=[
            pltpu.VMEM((nchunk, CH), jnp.int32),
            pltpu.VMEM((CH, H), jnp.float32),
            pltpu.VMEM_SHARED((NPAD, H), jnp.float32),
        ],
        compiler_params=pltpu.CompilerParams(use_tc_tiling_on_sc=False, needs_layout_passes=False),
    )
    return f(m, dst2d, z2d)


def _make_cpath_body(nchunk):
  tpb = nchunk * CH

  def _sc_cpath_body(cvec2d, dst2d, z1, cw, didx_v, c_v, cacc_v):
    c = lax.axis_index("c")
    s = lax.axis_index("s")
    t = s * NC + c
    rowbase = t * nchunk
    ebase = t * tpb

    pltpu.sync_copy(z1, cacc_v)
    pltpu.sync_copy(dst2d.at[pl.ds(rowbase, nchunk)], didx_v)

    iota16 = lax.iota(jnp.int32, 16)

    def chunk(j, carry):
        pltpu.sync_copy(cvec2d.at[pl.ds((ebase + j * CH) * 4 // 128, CH * 4 // 128)], c_v)
        for e0 in range(0, CH, 16):
            didx16 = didx_v[j, pl.ds(e0, 16)]
            addr = didx16 * 4
            for k in range(4):
                fl = iota16 * 4 + (e0 * 4 + k)
                vals = plsc.load_gather(c_v, [fl >> 7, fl & 127])
                plsc.addupdate_scatter(cacc_v, [addr + k], vals)
        return carry

    lax.fori_loop(0, nchunk, chunk, 0)
    pltpu.sync_copy(cacc_v, cw.at[t])

  return _sc_cpath_body


def _sc_cpath(cvec2d, dst2d, z1, nchunk):
    mesh = plsc.VectorSubcoreMesh(core_axis_name="c", subcore_axis_name="s")
    f = pl.kernel(
        _make_cpath_body(nchunk),
        out_type=[
            jax.ShapeDtypeStruct((NW, NP4), jnp.float32),
        ],
        mesh=mesh,
        scratch_types=[
            pltpu.VMEM((nchunk, CH), jnp.int32),
            pltpu.VMEM((CH * 4 // 128, 128), jnp.float32),
            pltpu.VMEM((NP4,), jnp.float32),
        ],
        compiler_params=pltpu.CompilerParams(use_tc_tiling_on_sc=False, needs_layout_passes=False),
    )
    return f(cvec2d, dst2d, z1)


def _node_body(x_ref, ms_ref, deg_ref, coord_ref, pos_ref, w3a_ref, w3b_ref,
               b3_ref, w4_ref, b4_ref, xn_ref, pn_ref):
    inv = 1.0 / jnp.maximum(deg_ref[...], 1.0)
    ms = ms_ref[...] * inv
    pre = (jnp.dot(x_ref[...], w3a_ref[...], preferred_element_type=jnp.float32)
           + jnp.dot(ms, w3b_ref[...], preferred_element_type=jnp.float32)
           + b3_ref[...])
    xn_ref[...] = jnp.dot(_silu(pre), w4_ref[...], preferred_element_type=jnp.float32) + b4_ref[...]
    pn_ref[...] = pos_ref[...] + coord_ref[...] * inv


def _node_mlp(x, m_sum, deg, coord, pos, W3a, W3b, b3, W4, b4):
    return pl.pallas_call(
        _node_body,
        grid=(N // NB,),
        in_specs=[
            pl.BlockSpec((NB, D), lambda i: (i, 0)),
            pl.BlockSpec((NB, H), lambda i: (i, 0)),
            pl.BlockSpec((NB, 1), lambda i: (i, 0)),
            pl.BlockSpec((NB, 3), lambda i: (i, 0)),
            pl.BlockSpec((NB, 3), lambda i: (i, 0)),
            pl.BlockSpec((D, H), lambda i: (0, 0)),
            pl.BlockSpec((H, H), lambda i: (0, 0)),
            pl.BlockSpec((1, H), lambda i: (0, 0)),
            pl.BlockSpec((H, D), lambda i: (0, 0)),
            pl.BlockSpec((1, D), lambda i: (0, 0)),
        ],
        out_specs=[
            pl.BlockSpec((NB, D), lambda i: (i, 0)),
            pl.BlockSpec((NB, 3), lambda i: (i, 0)),
        ],
        out_shape=[
            jax.ShapeDtypeStruct((N, D), jnp.float32),
            jax.ShapeDtypeStruct((N, 3), jnp.float32),
        ],
    )(x, m_sum, deg, coord, pos, W3a, W3b, b3, W4, b4)


K = 2
EPS = EP // K
NCHS = NCHUNK // K
ROWS = EPS // CH


def kernel(x, pos, edge_index, edge_attr, W1, b1, W2, b2, W3, b3, W4, b4, W5, b5):
    src = edge_index[0]
    dst = edge_index[1]

    W1a = W1[:D]
    W1b = W1[D:2 * D]
    w1r = W1[2 * D:2 * D + 1]
    W1e = W1[2 * D + 1:]
    W3a = W3[:D]
    W3b = W3[D:]

    xa, xb = _proj(x, W1a, W1b, b1.reshape(1, H))

    pad = EP - E
    zpad_i = jnp.zeros((pad,), jnp.int32)
    src_g2d = jnp.concatenate([src, zpad_i]).reshape(EP // CH, CH)
    dst_g2d = jnp.concatenate([dst, zpad_i]).reshape(EP // CH, CH)
    dst_s2d = jnp.concatenate([dst, jnp.full((pad,), N, jnp.int32)]).reshape(EP // CH, CH)
    ea_pad = jnp.concatenate([edge_attr, jnp.zeros((pad, ED), jnp.float32)])

    px = pos[:, 0]
    py = pos[:, 1]
    pz = pos[:, 2]
    z2d = jnp.zeros((NPAD, H), jnp.float32)
    z1 = jnp.zeros((NP4,), jnp.float32)

    maccs = []
    cws = []
    for k in range(K):
        sl = slice(k * ROWS, (k + 1) * ROWS)
        esl = slice(k * EPS, (k + 1) * EPS)
        g, r2f, dxf, dyf, dzf = _sc_gather(xa, xb, src_g2d[sl], dst_g2d[sl],
                                           px, py, pz, NCHS)
        m, cvec = _edge_mlp(g, r2f.reshape(EPS, 1), dxf.reshape(EPS, 1),
                            dyf.reshape(EPS, 1), dzf.reshape(EPS, 1),
                            ea_pad[esl], W1e, w1r, W2, b2.reshape(1, H),
                            W5, b5.reshape(1, 1), EPS)
        maccs.append(_sc_scatter(m, dst_s2d[sl], z2d, NCHS)[0])
        cws.append(_sc_cpath(cvec.reshape(EPS * 4 // 128, 128), dst_s2d[sl],
                             z1, NCHS)[0])

    m_sum = maccs[0][0, :N] + maccs[0][1, :N]
    cw = cws[0]
    for k in range(1, K):
        m_sum = m_sum + maccs[k][0, :N] + maccs[k][1, :N]
        cw = cw + cws[k]
    cs = cw.sum(axis=0).reshape(NPAD, 4)[:N]
    deg = cs[:, :1]
    coord = cs[:, 1:4]

    x_new, pos_new = _node_mlp(x, m_sum, deg, coord, pos, W3a, W3b,
                               b3.reshape(1, H), W4, b4.reshape(1, D))
    return (x_new, pos_new)


# speedup vs baseline: 3.0688x; 1.0382x over previous
import functools

import jax
import jax.numpy as jnp
from jax import lax
from jax.experimental import pallas as pl
from jax.experimental.pallas import tpu as pltpu
from jax.experimental.pallas import tpu_sc as plsc

N = 10000
E = 320000
D = 128
ED = 16
H = 128

EB = 1280
NB = 2000

NC = 2
NS = 16
NW = NC * NS
CH = 128
NCHUNK = 80
TPB = CH * NCHUNK
EP = NW * TPB
NPAD = 10112
NPS = NPAD // NS


def _silu(v):
    return v * (1.0 / (1.0 + jnp.exp(-v)))


def _proj_body(x_ref, w1a_ref, w1b_ref, b1_ref, xa_ref, xb_ref):
    x = x_ref[...]
    xa_ref[...] = jnp.dot(x, w1a_ref[...], preferred_element_type=jnp.float32) + b1_ref[...]
    xb_ref[...] = jnp.dot(x, w1b_ref[...], preferred_element_type=jnp.float32)


def _proj(x, W1a, W1b, b1):
    return pl.pallas_call(
        _proj_body,
        grid=(N // NB,),
        in_specs=[
            pl.BlockSpec((NB, D), lambda i: (i, 0)),
            pl.BlockSpec((D, H), lambda i: (0, 0)),
            pl.BlockSpec((D, H), lambda i: (0, 0)),
            pl.BlockSpec((1, H), lambda i: (0, 0)),
        ],
        out_specs=[
            pl.BlockSpec((NB, H), lambda i: (i, 0)),
            pl.BlockSpec((NB, H), lambda i: (i, 0)),
        ],
        out_shape=[
            jax.ShapeDtypeStruct((N, H), jnp.float32),
            jax.ShapeDtypeStruct((N, H), jnp.float32),
        ],
    )(x, W1a, W1b, b1)


def _make_gather_body(nchunk):
  niter = nchunk // 2
  tpb = nchunk * CH

  def _sc_gather_body(xa, xb, src2d, dst2d, px, py, pz,
                      g_out, r2_out, dx_out, dy_out, dz_out,
                      sidx_v, didx_v, px_v, py_v, pz_v, g_v0, g_v1,
                      r2_v, dx_v, dy_v, dz_v,
                      sem_a0, sem_a1, sem_b0, sem_b1, sem_w0, sem_w1):
    c = lax.axis_index("c")
    s = lax.axis_index("s")
    t = s * NC + c
    rowbase = t * nchunk
    ebase = t * tpb

    pltpu.sync_copy(src2d.at[pl.ds(rowbase, nchunk)], sidx_v)
    pltpu.sync_copy(dst2d.at[pl.ds(rowbase, nchunk)], didx_v)
    pltpu.sync_copy(px, px_v)
    pltpu.sync_copy(py, py_v)
    pltpu.sync_copy(pz, pz_v)

    def geometry(j):
        for k in range(CH // 16):
            off = j * CH + k * 16
            di = didx_v[j, pl.ds(k * 16, 16)]
            si = sidx_v[j, pl.ds(k * 16, 16)]
            dx = plsc.load_gather(px_v, [di]) - plsc.load_gather(px_v, [si])
            dy = plsc.load_gather(py_v, [di]) - plsc.load_gather(py_v, [si])
            dz = plsc.load_gather(pz_v, [di]) - plsc.load_gather(pz_v, [si])
            r2_v[pl.ds(off, 16)] = dx * dx + dy * dy + dz * dz
            dx_v[pl.ds(off, 16)] = dx
            dy_v[pl.ds(off, 16)] = dy
            dz_v[pl.ds(off, 16)] = dz

    def wait(src_ref, dst_ref, sem):
        pltpu.make_async_copy(src_ref, dst_ref, sem).wait()

    pltpu.async_copy(xa.at[didx_v.at[0]], g_v0, sem_a0)

    def body(jj, carry):
        a = 2 * jj
        b = 2 * jj + 1
        wait(xa.at[didx_v.at[a]], g_v0, sem_a0)
        pltpu.async_copy(xb.at[sidx_v.at[a]], g_v0, sem_b0, add=True)

        @pl.when(jj > 0)
        def _():
            wait(g_v1, g_out.at[pl.ds(ebase + (a - 1) * CH, CH)], sem_w1)

        pltpu.async_copy(xa.at[didx_v.at[b]], g_v1, sem_a1)

        geometry(a)

        wait(xb.at[sidx_v.at[a]], g_v0, sem_b0)
        pltpu.async_copy(g_v0, g_out.at[pl.ds(ebase + a * CH, CH)], sem_w0)

        wait(xa.at[didx_v.at[b]], g_v1, sem_a1)
        pltpu.async_copy(xb.at[sidx_v.at[b]], g_v1, sem_b1, add=True)

        geometry(b)

        wait(xb.at[sidx_v.at[b]], g_v1, sem_b1)
        pltpu.async_copy(g_v1, g_out.at[pl.ds(ebase + b * CH, CH)], sem_w1)

        @pl.when(jj + 1 < niter)
        def _():
            wait(g_v0, g_out.at[pl.ds(ebase + a * CH, CH)], sem_w0)
            pltpu.async_copy(xa.at[didx_v.at[a + 2]], g_v0, sem_a0)

        return carry

    lax.fori_loop(0, niter, body, 0)

    wait(g_v0, g_out.at[pl.ds(ebase, CH)], sem_w0)
    wait(g_v1, g_out.at[pl.ds(ebase, CH)], sem_w1)

    pltpu.sync_copy(r2_v, r2_out.at[pl.ds(ebase, tpb)])
    pltpu.sync_copy(dx_v, dx_out.at[pl.ds(ebase, tpb)])
    pltpu.sync_copy(dy_v, dy_out.at[pl.ds(ebase, tpb)])
    pltpu.sync_copy(dz_v, dz_out.at[pl.ds(ebase, tpb)])

  return _sc_gather_body


def _sc_gather(xa, xb, src2d, dst2d, px, py, pz, nchunk):
    ep = NW * nchunk * CH
    tpb = nchunk * CH
    mesh = plsc.VectorSubcoreMesh(core_axis_name="c", subcore_axis_name="s")
    f = pl.kernel(
        _make_gather_body(nchunk),
        out_type=[
            jax.ShapeDtypeStruct((ep, H), jnp.float32),
            jax.ShapeDtypeStruct((ep,), jnp.float32),
            jax.ShapeDtypeStruct((ep,), jnp.float32),
            jax.ShapeDtypeStruct((ep,), jnp.float32),
            jax.ShapeDtypeStruct((ep,), jnp.float32),
        ],
        mesh=mesh,
        scratch_types=[
            pltpu.VMEM((nchunk, CH), jnp.int32),
            pltpu.VMEM((nchunk, CH), jnp.int32),
            pltpu.VMEM((N,), jnp.float32),
            pltpu.VMEM((N,), jnp.float32),
            pltpu.VMEM((N,), jnp.float32),
            pltpu.VMEM((CH, H), jnp.float32),
            pltpu.VMEM((CH, H), jnp.float32),
            pltpu.VMEM((tpb,), jnp.float32),
            pltpu.VMEM((tpb,), jnp.float32),
            pltpu.VMEM((tpb,), jnp.float32),
            pltpu.VMEM((tpb,), jnp.float32),
            pltpu.SemaphoreType.DMA,
            pltpu.SemaphoreType.DMA,
            pltpu.SemaphoreType.DMA,
            pltpu.SemaphoreType.DMA,
            pltpu.SemaphoreType.DMA,
            pltpu.SemaphoreType.DMA,
        ],
        compiler_params=pltpu.CompilerParams(use_tc_tiling_on_sc=False, needs_layout_passes=False),
    )
    return f(xa, xb, src2d, dst2d, px, py, pz)


def _edge_body(g_ref, r2_ref, dx_ref, dy_ref, dz_ref, ea_ref, w1e_ref,
               w1r_ref, w2_ref, b2_ref, w5_ref, b5_ref, m_ref, cv_ref):
    r2 = r2_ref[...]
    pre1 = (g_ref[...]
            + r2 * w1r_ref[...]
            + jnp.dot(ea_ref[...], w1e_ref[...], preferred_element_type=jnp.float32))
    z1 = _silu(pre1)
    m = _silu(jnp.dot(z1, w2_ref[...], preferred_element_type=jnp.float32) + b2_ref[...])
    m_ref[...] = m
    gamma = jnp.dot(m, w5_ref[...], preferred_element_type=jnp.float32) + b5_ref[...]
    sc = gamma * lax.rsqrt(r2 + 1e-08)
    ones = jnp.ones_like(sc)
    cv_ref[...] = jnp.concatenate(
        [ones, sc * dx_ref[...], sc * dy_ref[...], sc * dz_ref[...]], axis=1)


def _edge_mlp(g, r2, dx, dy, dz, ea, W1e, w1r, W2, b2, W5, b5, ep):
    return pl.pallas_call(
        _edge_body,
        grid=(ep // EB,),
        in_specs=[
            pl.BlockSpec((EB, H), lambda i: (i, 0)),
            pl.BlockSpec((EB, 1), lambda i: (i, 0)),
            pl.BlockSpec((EB, 1), lambda i: (i, 0)),
            pl.BlockSpec((EB, 1), lambda i: (i, 0)),
            pl.BlockSpec((EB, 1), lambda i: (i, 0)),
            pl.BlockSpec((EB, ED), lambda i: (i, 0)),
            pl.BlockSpec((ED, H), lambda i: (0, 0)),
            pl.BlockSpec((1, H), lambda i: (0, 0)),
            pl.BlockSpec((H, H), lambda i: (0, 0)),
            pl.BlockSpec((1, H), lambda i: (0, 0)),
            pl.BlockSpec((H, 1), lambda i: (0, 0)),
            pl.BlockSpec((1, 1), lambda i: (0, 0)),
        ],
        out_specs=[
            pl.BlockSpec((EB, H), lambda i: (i, 0)),
            pl.BlockSpec((EB, 4), lambda i: (i, 0)),
        ],
        out_shape=[
            jax.ShapeDtypeStruct((ep, H), jnp.float32),
            jax.ShapeDtypeStruct((ep, 4), jnp.float32),
        ],
        compiler_params=pltpu.CompilerParams(
            dimension_semantics=("arbitrary",),
        ),
    )(g, r2, dx, dy, dz, ea, W1e, w1r, W2, b2, W5, b5)


CV2D = EP * 4 // 128
NP4 = NPAD * 4


def _make_scatter_body(nchunk):
  tpb = nchunk * CH

  def _sc_scatter_body(m, dst2d, z2d, macc,
                       didx_v, m_v, m_sh):
    c = lax.axis_index("c")
    s = lax.axis_index("s")
    t = s * NC + c
    rowbase = t * nchunk
    ebase = t * tpb

    pltpu.sync_copy(z2d.at[pl.ds(s * NPS, NPS)], m_sh.at[pl.ds(s * NPS, NPS)])
    pltpu.sync_copy(dst2d.at[pl.ds(rowbase, nchunk)], didx_v)
    plsc.subcore_barrier()

    def chunk(j, carry):
        pltpu.sync_copy(m.at[pl.ds(ebase + j * CH, CH)], m_v)
        pltpu.sync_copy(m_v, m_sh.at[didx_v.at[j]], add=True)
        return carry

    lax.fori_loop(0, nchunk, chunk, 0)
    plsc.subcore_barrier()

    pltpu.sync_copy(m_sh.at[pl.ds(s * NPS, NPS)], macc.at[c, pl.ds(s * NPS, NPS)])

  return _sc_scatter_body


def _sc_scatter(m, dst2d, z2d, nchunk):
    mesh = plsc.VectorSubcoreMesh(core_axis_name="c", subcore_axis_name="s")
    f = pl.kernel(
        _make_scatter_body(nchunk),
        out_type=[
            jax.ShapeDtypeStruct((NC, NPAD, H), jnp.float32),
        ],
        mesh=mesh,
        scratch_types=[
            pltpu.VMEM((nchunk, CH), jnp.int32),
            pltpu.VMEM((CH, H), jnp.float32),
            pltpu.VMEM_SHARED((NPAD, H), jnp.float32),
        ],
        compiler_params=pltpu.CompilerParams(use_tc_tiling_on_sc=False, needs_layout_passes=False),
    )
    return f(m, dst2d, z2d)


def _make_cpath_body(nchunk):
  tpb = nchunk * CH

  def _sc_cpath_body(cvec2d, dst2d, z1, cw, didx_v, c_v, cacc_v):
    c = lax.axis_index("c")
    s = lax.axis_index("s")
    t = s * NC + c
    rowbase = t * nchunk
    ebase = t * tpb

    pltpu.sync_copy(z1, cacc_v)
    pltpu.sync_copy(dst2d.at[pl.ds(rowbase, nchunk)], didx_v)

    iota16 = lax.iota(jnp.int32, 16)

    def chunk(j, carry):
        pltpu.sync_copy(cvec2d.at[pl.ds((ebase + j * CH) * 4 // 128, CH * 4 // 128)], c_v)
        for e0 in range(0, CH, 16):
            didx16 = didx_v[j, pl.ds(e0, 16)]
            addr = didx16 * 4
            for k in range(4):
                fl = iota16 * 4 + (e0 * 4 + k)
                vals = plsc.load_gather(c_v, [fl >> 7, fl & 127])
                plsc.addupdate_scatter(cacc_v, [addr + k], vals)
        return carry

    lax.fori_loop(0, nchunk, chunk, 0)
    pltpu.sync_copy(cacc_v, cw.at[t])

  return _sc_cpath_body


def _sc_cpath(cvec2d, dst2d, z1, nchunk):
    mesh = plsc.VectorSubcoreMesh(core_axis_name="c", subcore_axis_name="s")
    f = pl.kernel(
        _make_cpath_body(nchunk),
        out_type=[
            jax.ShapeDtypeStruct((NW, NP4), jnp.float32),
        ],
        mesh=mesh,
        scratch_types=[
            pltpu.VMEM((nchunk, CH), jnp.int32),
            pltpu.VMEM((CH * 4 // 128, 128), jnp.float32),
            pltpu.VMEM((NP4,), jnp.float32),
        ],
        compiler_params=pltpu.CompilerParams(use_tc_tiling_on_sc=False, needs_layout_passes=False),
    )
    return f(cvec2d, dst2d, z1)


def _node_body(x_ref, ms_ref, deg_ref, coord_ref, pos_ref, w3a_ref, w3b_ref,
               b3_ref, w4_ref, b4_ref, xn_ref, pn_ref):
    inv = 1.0 / jnp.maximum(deg_ref[...], 1.0)
    ms = ms_ref[...] * inv
    pre = (jnp.dot(x_ref[...], w3a_ref[...], preferred_element_type=jnp.float32)
           + jnp.dot(ms, w3b_ref[...], preferred_element_type=jnp.float32)
           + b3_ref[...])
    xn_ref[...] = jnp.dot(_silu(pre), w4_ref[...], preferred_element_type=jnp.float32) + b4_ref[...]
    pn_ref[...] = pos_ref[...] + coord_ref[...] * inv


def _node_mlp(x, m_sum, deg, coord, pos, W3a, W3b, b3, W4, b4):
    return pl.pallas_call(
        _node_body,
        grid=(N // NB,),
        in_specs=[
            pl.BlockSpec((NB, D), lambda i: (i, 0)),
            pl.BlockSpec((NB, H), lambda i: (i, 0)),
            pl.BlockSpec((NB, 1), lambda i: (i, 0)),
            pl.BlockSpec((NB, 3), lambda i: (i, 0)),
            pl.BlockSpec((NB, 3), lambda i: (i, 0)),
            pl.BlockSpec((D, H), lambda i: (0, 0)),
            pl.BlockSpec((H, H), lambda i: (0, 0)),
            pl.BlockSpec((1, H), lambda i: (0, 0)),
            pl.BlockSpec((H, D), lambda i: (0, 0)),
            pl.BlockSpec((1, D), lambda i: (0, 0)),
        ],
        out_specs=[
            pl.BlockSpec((NB, D), lambda i: (i, 0)),
            pl.BlockSpec((NB, 3), lambda i: (i, 0)),
        ],
        out_shape=[
            jax.ShapeDtypeStruct((N, D), jnp.float32),
            jax.ShapeDtypeStruct((N, 3), jnp.float32),
        ],
    )(x, m_sum, deg, coord, pos, W3a, W3b, b3, W4, b4)


K = 4
EPS = EP // K
NCHS = NCHUNK // K
ROWS = EPS // CH


def kernel(x, pos, edge_index, edge_attr, W1, b1, W2, b2, W3, b3, W4, b4, W5, b5):
    src = edge_index[0]
    dst = edge_index[1]

    W1a = W1[:D]
    W1b = W1[D:2 * D]
    w1r = W1[2 * D:2 * D + 1]
    W1e = W1[2 * D + 1:]
    W3a = W3[:D]
    W3b = W3[D:]

    xa, xb = _proj(x, W1a, W1b, b1.reshape(1, H))

    pad = EP - E
    zpad_i = jnp.zeros((pad,), jnp.int32)
    src_g2d = jnp.concatenate([src, zpad_i]).reshape(EP // CH, CH)
    dst_g2d = jnp.concatenate([dst, zpad_i]).reshape(EP // CH, CH)
    dst_s2d = jnp.concatenate([dst, jnp.full((pad,), N, jnp.int32)]).reshape(EP // CH, CH)
    ea_pad = jnp.concatenate([edge_attr, jnp.zeros((pad, ED), jnp.float32)])

    px = pos[:, 0]
    py = pos[:, 1]
    pz = pos[:, 2]
    z2d = jnp.zeros((NPAD, H), jnp.float32)
    z1 = jnp.zeros((NP4,), jnp.float32)

    maccs = []
    cws = []
    for k in range(K):
        sl = slice(k * ROWS, (k + 1) * ROWS)
        esl = slice(k * EPS, (k + 1) * EPS)
        g, r2f, dxf, dyf, dzf = _sc_gather(xa, xb, src_g2d[sl], dst_g2d[sl],
                                           px, py, pz, NCHS)
        m, cvec = _edge_mlp(g, r2f.reshape(EPS, 1), dxf.reshape(EPS, 1),
                            dyf.reshape(EPS, 1), dzf.reshape(EPS, 1),
                            ea_pad[esl], W1e, w1r, W2, b2.reshape(1, H),
                            W5, b5.reshape(1, 1), EPS)
        maccs.append(_sc_scatter(m, dst_s2d[sl], z2d, NCHS)[0])
        cws.append(_sc_cpath(cvec.reshape(EPS * 4 // 128, 128), dst_s2d[sl],
                             z1, NCHS)[0])

    m_sum = maccs[0][0, :N] + maccs[0][1, :N]
    cw = cws[0]
    for k in range(1, K):
        m_sum = m_sum + maccs[k][0, :N] + maccs[k][1, :N]
        cw = cw + cws[k]
    cs = cw.sum(axis=0).reshape(NPAD, 4)[:N]
    deg = cs[:, :1]
    coord = cs[:, 1:4]

    x_new, pos_new = _node_mlp(x, m_sum, deg, coord, pos, W3a, W3b,
                               b3.reshape(1, H), W4, b4.reshape(1, D))
    return (x_new, pos_new)


# speedup vs baseline: 3.1050x; 1.0118x over previous
import jax
import jax.numpy as jnp
from jax import lax
from jax.experimental import pallas as pl
from jax.experimental.pallas import tpu as pltpu
from jax.experimental.pallas import tpu_sc as plsc

N = 10000
E = 320000
D = 128
ED = 16
H = 128

EB = 1280
NB = 2000

NC = 2
NS = 16
NW = NC * NS
CH = 128
NCHUNK = 80
TPB = CH * NCHUNK
EP = NW * TPB
NPAD = 10112
NPS = NPAD // NS


def _silu(v):
    return v * (1.0 / (1.0 + jnp.exp(-v)))


def _proj_body(x_ref, w1a_ref, w1b_ref, b1_ref, xa_ref, xb_ref):
    x = x_ref[...]
    xa_ref[...] = jnp.dot(x, w1a_ref[...], preferred_element_type=jnp.float32) + b1_ref[...]
    xb_ref[...] = jnp.dot(x, w1b_ref[...], preferred_element_type=jnp.float32)


def _proj(x, W1a, W1b, b1):
    return pl.pallas_call(
        _proj_body,
        grid=(N // NB,),
        in_specs=[
            pl.BlockSpec((NB, D), lambda i: (i, 0)),
            pl.BlockSpec((D, H), lambda i: (0, 0)),
            pl.BlockSpec((D, H), lambda i: (0, 0)),
            pl.BlockSpec((1, H), lambda i: (0, 0)),
        ],
        out_specs=[
            pl.BlockSpec((NB, H), lambda i: (i, 0)),
            pl.BlockSpec((NB, H), lambda i: (i, 0)),
        ],
        out_shape=[
            jax.ShapeDtypeStruct((N, H), jnp.float32),
            jax.ShapeDtypeStruct((N, H), jnp.float32),
        ],
    )(x, W1a, W1b, b1)


def _make_gather_body(nchunk):
  niter = nchunk // 2
  tpb = nchunk * CH

  def _sc_gather_body(xa, xb, src2d, dst2d, px, py, pz,
                      g_out, r2_out, dx_out, dy_out, dz_out,
                      sidx_v, didx_v, px_v, py_v, pz_v, g_v0, g_v1,
                      r2_v, dx_v, dy_v, dz_v,
                      sem_a0, sem_a1, sem_b0, sem_b1, sem_w0, sem_w1):
    c = lax.axis_index("c")
    s = lax.axis_index("s")
    t = s * NC + c
    rowbase = t * nchunk
    ebase = t * tpb

    pltpu.sync_copy(src2d.at[pl.ds(rowbase, nchunk)], sidx_v)
    pltpu.sync_copy(dst2d.at[pl.ds(rowbase, nchunk)], didx_v)
    pltpu.sync_copy(px, px_v)
    pltpu.sync_copy(py, py_v)
    pltpu.sync_copy(pz, pz_v)

    def geometry(j):
        for k in range(CH // 16):
            off = j * CH + k * 16
            di = didx_v[j, pl.ds(k * 16, 16)]
            si = sidx_v[j, pl.ds(k * 16, 16)]
            dx = plsc.load_gather(px_v, [di]) - plsc.load_gather(px_v, [si])
            dy = plsc.load_gather(py_v, [di]) - plsc.load_gather(py_v, [si])
            dz = plsc.load_gather(pz_v, [di]) - plsc.load_gather(pz_v, [si])
            r2_v[pl.ds(off, 16)] = dx * dx + dy * dy + dz * dz
            dx_v[pl.ds(off, 16)] = dx
            dy_v[pl.ds(off, 16)] = dy
            dz_v[pl.ds(off, 16)] = dz

    def wait(src_ref, dst_ref, sem):
        pltpu.make_async_copy(src_ref, dst_ref, sem).wait()

    pltpu.async_copy(xa.at[didx_v.at[0]], g_v0, sem_a0)

    def body(jj, carry):
        a = 2 * jj
        b = 2 * jj + 1
        wait(xa.at[didx_v.at[a]], g_v0, sem_a0)
        pltpu.async_copy(xb.at[sidx_v.at[a]], g_v0, sem_b0, add=True)

        @pl.when(jj > 0)
        def _():
            wait(g_v1, g_out.at[pl.ds(ebase + (a - 1) * CH, CH)], sem_w1)

        pltpu.async_copy(xa.at[didx_v.at[b]], g_v1, sem_a1)

        geometry(a)

        wait(xb.at[sidx_v.at[a]], g_v0, sem_b0)
        pltpu.async_copy(g_v0, g_out.at[pl.ds(ebase + a * CH, CH)], sem_w0)

        wait(xa.at[didx_v.at[b]], g_v1, sem_a1)
        pltpu.async_copy(xb.at[sidx_v.at[b]], g_v1, sem_b1, add=True)

        geometry(b)

        wait(xb.at[sidx_v.at[b]], g_v1, sem_b1)
        pltpu.async_copy(g_v1, g_out.at[pl.ds(ebase + b * CH, CH)], sem_w1)

        @pl.when(jj + 1 < niter)
        def _():
            wait(g_v0, g_out.at[pl.ds(ebase + a * CH, CH)], sem_w0)
            pltpu.async_copy(xa.at[didx_v.at[a + 2]], g_v0, sem_a0)

        return carry

    lax.fori_loop(0, niter, body, 0)

    wait(g_v0, g_out.at[pl.ds(ebase, CH)], sem_w0)
    wait(g_v1, g_out.at[pl.ds(ebase, CH)], sem_w1)

    pltpu.sync_copy(r2_v, r2_out.at[pl.ds(ebase, tpb)])
    pltpu.sync_copy(dx_v, dx_out.at[pl.ds(ebase, tpb)])
    pltpu.sync_copy(dy_v, dy_out.at[pl.ds(ebase, tpb)])
    pltpu.sync_copy(dz_v, dz_out.at[pl.ds(ebase, tpb)])

  return _sc_gather_body


def _sc_gather(xa, xb, src2d, dst2d, px, py, pz, nchunk):
    ep = NW * nchunk * CH
    tpb = nchunk * CH
    mesh = plsc.VectorSubcoreMesh(core_axis_name="c", subcore_axis_name="s")
    f = pl.kernel(
        _make_gather_body(nchunk),
        out_type=[
            jax.ShapeDtypeStruct((ep, H), jnp.float32),
            jax.ShapeDtypeStruct((ep,), jnp.float32),
            jax.ShapeDtypeStruct((ep,), jnp.float32),
            jax.ShapeDtypeStruct((ep,), jnp.float32),
            jax.ShapeDtypeStruct((ep,), jnp.float32),
        ],
        mesh=mesh,
        scratch_types=[
            pltpu.VMEM((nchunk, CH), jnp.int32),
            pltpu.VMEM((nchunk, CH), jnp.int32),
            pltpu.VMEM((N,), jnp.float32),
            pltpu.VMEM((N,), jnp.float32),
            pltpu.VMEM((N,), jnp.float32),
            pltpu.VMEM((CH, H), jnp.float32),
            pltpu.VMEM((CH, H), jnp.float32),
            pltpu.VMEM((tpb,), jnp.float32),
            pltpu.VMEM((tpb,), jnp.float32),
            pltpu.VMEM((tpb,), jnp.float32),
            pltpu.VMEM((tpb,), jnp.float32),
            pltpu.SemaphoreType.DMA,
            pltpu.SemaphoreType.DMA,
            pltpu.SemaphoreType.DMA,
            pltpu.SemaphoreType.DMA,
            pltpu.SemaphoreType.DMA,
            pltpu.SemaphoreType.DMA,
        ],
        compiler_params=pltpu.CompilerParams(use_tc_tiling_on_sc=False, needs_layout_passes=False),
    )
    return f(xa, xb, src2d, dst2d, px, py, pz)


def _edge_body(g_ref, r2_ref, dx_ref, dy_ref, dz_ref, ea_ref, w1e_ref,
               w1r_ref, w2_ref, b2_ref, w5_ref, b5_ref, m_ref, cv_ref):
    r2 = r2_ref[...]
    pre1 = (g_ref[...]
            + r2 * w1r_ref[...]
            + jnp.dot(ea_ref[...], w1e_ref[...], preferred_element_type=jnp.float32))
    z1 = _silu(pre1)
    m = _silu(jnp.dot(z1, w2_ref[...], preferred_element_type=jnp.float32) + b2_ref[...])
    m_ref[...] = m
    gamma = jnp.dot(m, w5_ref[...], preferred_element_type=jnp.float32) + b5_ref[...]
    sc = gamma * lax.rsqrt(r2 + 1e-08)
    ones = jnp.ones_like(sc)
    cv_ref[...] = jnp.concatenate(
        [ones, sc * dx_ref[...], sc * dy_ref[...], sc * dz_ref[...]], axis=1)


def _edge_mlp(g, r2, dx, dy, dz, ea, W1e, w1r, W2, b2, W5, b5, ep):
    return pl.pallas_call(
        _edge_body,
        grid=(ep // EB,),
        in_specs=[
            pl.BlockSpec((EB, H), lambda i: (i, 0)),
            pl.BlockSpec((EB, 1), lambda i: (i, 0)),
            pl.BlockSpec((EB, 1), lambda i: (i, 0)),
            pl.BlockSpec((EB, 1), lambda i: (i, 0)),
            pl.BlockSpec((EB, 1), lambda i: (i, 0)),
            pl.BlockSpec((EB, ED), lambda i: (i, 0)),
            pl.BlockSpec((ED, H), lambda i: (0, 0)),
            pl.BlockSpec((1, H), lambda i: (0, 0)),
            pl.BlockSpec((H, H), lambda i: (0, 0)),
            pl.BlockSpec((1, H), lambda i: (0, 0)),
            pl.BlockSpec((H, 1), lambda i: (0, 0)),
            pl.BlockSpec((1, 1), lambda i: (0, 0)),
        ],
        out_specs=[
            pl.BlockSpec((EB, H), lambda i: (i, 0)),
            pl.BlockSpec((EB, 4), lambda i: (i, 0)),
        ],
        out_shape=[
            jax.ShapeDtypeStruct((ep, H), jnp.float32),
            jax.ShapeDtypeStruct((ep, 4), jnp.float32),
        ],
        compiler_params=pltpu.CompilerParams(
            dimension_semantics=("arbitrary",),
        ),
    )(g, r2, dx, dy, dz, ea, W1e, w1r, W2, b2, W5, b5)


CV2D = EP * 4 // 128
NP4 = NPAD * 4


def _make_scatter_body(nchunk):
  tpb = nchunk * CH

  def _sc_scatter_body(m, dst2d, z2d, macc,
                       didx_v, m_v, m_sh):
    c = lax.axis_index("c")
    s = lax.axis_index("s")
    t = s * NC + c
    rowbase = t * nchunk
    ebase = t * tpb

    pltpu.sync_copy(z2d.at[pl.ds(s * NPS, NPS)], m_sh.at[pl.ds(s * NPS, NPS)])
    pltpu.sync_copy(dst2d.at[pl.ds(rowbase, nchunk)], didx_v)
    plsc.subcore_barrier()

    def chunk(j, carry):
        pltpu.sync_copy(m.at[pl.ds(ebase + j * CH, CH)], m_v)
        pltpu.sync_copy(m_v, m_sh.at[didx_v.at[j]], add=True)
        return carry

    lax.fori_loop(0, nchunk, chunk, 0)
    plsc.subcore_barrier()

    pltpu.sync_copy(m_sh.at[pl.ds(s * NPS, NPS)], macc.at[c, pl.ds(s * NPS, NPS)])

  return _sc_scatter_body


def _sc_scatter(m, dst2d, z2d, nchunk):
    mesh = plsc.VectorSubcoreMesh(core_axis_name="c", subcore_axis_name="s")
    f = pl.kernel(
        _make_scatter_body(nchunk),
        out_type=[
            jax.ShapeDtypeStruct((NC, NPAD, H), jnp.float32),
        ],
        mesh=mesh,
        scratch_types=[
            pltpu.VMEM((nchunk, CH), jnp.int32),
            pltpu.VMEM((CH, H), jnp.float32),
            pltpu.VMEM_SHARED((NPAD, H), jnp.float32),
        ],
        compiler_params=pltpu.CompilerParams(use_tc_tiling_on_sc=False, needs_layout_passes=False),
    )
    return f(m, dst2d, z2d)


def _make_cpath_body(nchunk):
  tpb = nchunk * CH

  def _sc_cpath_body(cvec2d, dst2d, z1, cw, didx_v, c_v, cacc_v):
    c = lax.axis_index("c")
    s = lax.axis_index("s")
    t = s * NC + c
    rowbase = t * nchunk
    ebase = t * tpb

    pltpu.sync_copy(z1, cacc_v)
    pltpu.sync_copy(dst2d.at[pl.ds(rowbase, nchunk)], didx_v)

    iota16 = lax.iota(jnp.int32, 16)

    def chunk(j, carry):
        pltpu.sync_copy(cvec2d.at[pl.ds((ebase + j * CH) * 4 // 128, CH * 4 // 128)], c_v)
        for e0 in range(0, CH, 16):
            didx16 = didx_v[j, pl.ds(e0, 16)]
            addr = didx16 * 4
            for k in range(4):
                fl = iota16 * 4 + (e0 * 4 + k)
                vals = plsc.load_gather(c_v, [fl >> 7, fl & 127])
                plsc.addupdate_scatter(cacc_v, [addr + k], vals)
        return carry

    lax.fori_loop(0, nchunk, chunk, 0)
    pltpu.sync_copy(cacc_v, cw.at[t])

  return _sc_cpath_body


def _sc_cpath(cvec2d, dst2d, z1, nchunk):
    mesh = plsc.VectorSubcoreMesh(core_axis_name="c", subcore_axis_name="s")
    f = pl.kernel(
        _make_cpath_body(nchunk),
        out_type=[
            jax.ShapeDtypeStruct((NW, NP4), jnp.float32),
        ],
        mesh=mesh,
        scratch_types=[
            pltpu.VMEM((nchunk, CH), jnp.int32),
            pltpu.VMEM((CH * 4 // 128, 128), jnp.float32),
            pltpu.VMEM((NP4,), jnp.float32),
        ],
        compiler_params=pltpu.CompilerParams(use_tc_tiling_on_sc=False, needs_layout_passes=False),
    )
    return f(cvec2d, dst2d, z1)


def _node_body(x_ref, ms_ref, deg_ref, coord_ref, pos_ref, w3a_ref, w3b_ref,
               b3_ref, w4_ref, b4_ref, xn_ref, pn_ref):
    inv = 1.0 / jnp.maximum(deg_ref[...], 1.0)
    ms = ms_ref[...] * inv
    pre = (jnp.dot(x_ref[...], w3a_ref[...], preferred_element_type=jnp.float32)
           + jnp.dot(ms, w3b_ref[...], preferred_element_type=jnp.float32)
           + b3_ref[...])
    xn_ref[...] = jnp.dot(_silu(pre), w4_ref[...], preferred_element_type=jnp.float32) + b4_ref[...]
    pn_ref[...] = pos_ref[...] + coord_ref[...] * inv


def _node_mlp(x, m_sum, deg, coord, pos, W3a, W3b, b3, W4, b4):
    return pl.pallas_call(
        _node_body,
        grid=(N // NB,),
        in_specs=[
            pl.BlockSpec((NB, D), lambda i: (i, 0)),
            pl.BlockSpec((NB, H), lambda i: (i, 0)),
            pl.BlockSpec((NB, 1), lambda i: (i, 0)),
            pl.BlockSpec((NB, 3), lambda i: (i, 0)),
            pl.BlockSpec((NB, 3), lambda i: (i, 0)),
            pl.BlockSpec((D, H), lambda i: (0, 0)),
            pl.BlockSpec((H, H), lambda i: (0, 0)),
            pl.BlockSpec((1, H), lambda i: (0, 0)),
            pl.BlockSpec((H, D), lambda i: (0, 0)),
            pl.BlockSpec((1, D), lambda i: (0, 0)),
        ],
        out_specs=[
            pl.BlockSpec((NB, D), lambda i: (i, 0)),
            pl.BlockSpec((NB, 3), lambda i: (i, 0)),
        ],
        out_shape=[
            jax.ShapeDtypeStruct((N, D), jnp.float32),
            jax.ShapeDtypeStruct((N, 3), jnp.float32),
        ],
    )(x, m_sum, deg, coord, pos, W3a, W3b, b3, W4, b4)


K = 4
EPS = EP // K
NCHS = NCHUNK // K
ROWS = EPS // CH


def kernel(x, pos, edge_index, edge_attr, W1, b1, W2, b2, W3, b3, W4, b4, W5, b5):
    src = edge_index[0]
    dst = edge_index[1]

    W1a = W1[:D]
    W1b = W1[D:2 * D]
    w1r = W1[2 * D:2 * D + 1]
    W1e = W1[2 * D + 1:]
    W3a = W3[:D]
    W3b = W3[D:]

    xa, xb = _proj(x, W1a, W1b, b1.reshape(1, H))

    pad = EP - E
    zpad_i = jnp.zeros((pad,), jnp.int32)
    src_g2d = jnp.concatenate([src, zpad_i]).reshape(EP // CH, CH)
    dst_g2d = jnp.concatenate([dst, zpad_i]).reshape(EP // CH, CH)
    dst_s2d = jnp.concatenate([dst, jnp.full((pad,), N, jnp.int32)]).reshape(EP // CH, CH)
    ea_pad = jnp.concatenate([edge_attr, jnp.zeros((pad, ED), jnp.float32)])

    px = pos[:, 0]
    py = pos[:, 1]
    pz = pos[:, 2]
    z2d = jnp.zeros((NPAD, H), jnp.float32)
    z1 = jnp.zeros((NP4,), jnp.float32)

    maccs = []
    cws = []
    for k in range(K):
        sl = slice(k * ROWS, (k + 1) * ROWS)
        esl = slice(k * EPS, (k + 1) * EPS)
        g, r2f, dxf, dyf, dzf = _sc_gather(xa, xb, src_g2d[sl], dst_g2d[sl],
                                           px, py, pz, NCHS)
        m, cvec = _edge_mlp(g, r2f.reshape(EPS, 1), dxf.reshape(EPS, 1),
                            dyf.reshape(EPS, 1), dzf.reshape(EPS, 1),
                            ea_pad[esl], W1e, w1r, W2, b2.reshape(1, H),
                            W5, b5.reshape(1, 1), EPS)
        maccs.append(_sc_scatter(m, dst_s2d[sl], z2d, NCHS)[0])
        cws.append(_sc_cpath(cvec.reshape(EPS * 4 // 128, 128), dst_s2d[sl],
                             z1, NCHS)[0])

    m_sum = maccs[0][0, :N] + maccs[0][1, :N]
    cw = cws[0]
    for k in range(1, K):
        m_sum = m_sum + maccs[k][0, :N] + maccs[k][1, :N]
        cw = cw + cws[k]
    cs = cw.sum(axis=0).reshape(NPAD, 4)[:N]
    deg = cs[:, :1]
    coord = cs[:, 1:4]

    x_new, pos_new = _node_mlp(x, m_sum, deg, coord, pos, W3a, W3b,
                               b3.reshape(1, H), W4, b4.reshape(1, D))
    return (x_new, pos_new)
